# Initial kernel scaffold; baseline (speedup 1.0000x reference)
import jax
import jax.numpy as jnp
from jax.experimental import pallas as pl


def kernel(ind_1, elems, coord, ind_2, dist, diff, params):
    raise NotImplementedError("write your pallas kernel here")


# speedup vs baseline: 15.7197x; 15.7197x over previous
import functools

import numpy as np
import jax
import jax.numpy as jnp
from jax import lax
from jax.experimental import pallas as pl
from jax.experimental.pallas import tpu as pltpu
import jax.experimental.pallas.tpu_sc as plsc

C = 16
NB = 4
NE = 4
DEPTH = 4
RC = 4.0
NA = 50000
NP = 800000
D = 144

BP = 2000
BA = 2000
NWORK = 32
PCHUNK = NP // NWORK
GW = 200
NWIN = PCHUNK // GW
RPT = NA // 16
ZR = 125

_f32 = jnp.float32


def _full_spec(shape):
    n = len(shape)
    return pl.BlockSpec(shape, lambda i: (0,) * n)


def _ssq(diff):
    bs = 32000

    def body(d_ref, o_ref):
        @pl.when(pl.program_id(0) == 0)
        def _():
            o_ref[0, 0] = 0.0

        d = d_ref[...]
        o_ref[0, 0] += jnp.sum(d * d)

    return pl.pallas_call(
        body,
        grid=(NP // bs,),
        in_specs=[pl.BlockSpec((bs, 3), lambda i: (i, 0))],
        out_specs=pl.BlockSpec((1, 1), lambda i: (0, 0),
                               memory_space=pltpu.SMEM),
        out_shape=jax.ShapeDtypeStruct((1, 1), _f32),
    )(diff)


def _onehot(elems2d):
    def body(e_ref, t_ref):
        e = e_ref[...]
        cols = [(e == t).astype(_f32) for t in (1, 6, 7, 8)]
        cols.append(jnp.zeros((e.shape[0], 12), _f32))
        t_ref[...] = jnp.concatenate(cols, axis=1)

    return pl.pallas_call(
        body,
        grid=(NA // BA,),
        in_specs=[pl.BlockSpec((BA, 1), lambda i: (i, 0))],
        out_specs=pl.BlockSpec((BA, 16), lambda i: (i, 0)),
        out_shape=jax.ShapeDtypeStruct((NA, 16), _f32),
    )(elems2d)


def _make_gather(dg):
    mesh = plsc.VectorSubcoreMesh(core_axis_name="c", subcore_axis_name="s")

    @functools.partial(
        pl.kernel,
        mesh=mesh,
        out_type=(
            jax.ShapeDtypeStruct((NP, dg), _f32),
            jax.ShapeDtypeStruct((NP, dg), _f32),
        ),
        scratch_types=(
            pltpu.VMEM((GW,), jnp.int32),
            pltpu.VMEM((GW,), jnp.int32),
            pltpu.VMEM((GW, dg), _f32),
            pltpu.VMEM((GW, dg), _f32),
            pltpu.SemaphoreType.DMA,
            pltpu.SemaphoreType.DMA,
        ),
        compiler_params=pltpu.CompilerParams(use_tc_tiling_on_sc=False),
        name=f"pair_gather_{dg}",
    )
    def k(ti, tj, ii, ij, gi, gj, iiv, ijv, ri, rj, s1, s2):
        wid = lax.axis_index("s") * 2 + lax.axis_index("c")
        base = wid * PCHUNK

        def body(w, carry):
            start = base + w * GW
            pltpu.sync_copy(ii.at[pl.ds(start, GW)], iiv)
            pltpu.sync_copy(ij.at[pl.ds(start, GW)], ijv)
            c1 = pltpu.async_copy(ti.at[iiv], ri, s1)
            c2 = pltpu.async_copy(tj.at[ijv], rj, s2)
            c1.wait()
            c2.wait()
            pltpu.sync_copy(ri, gi.at[pl.ds(start, GW)])
            pltpu.sync_copy(rj, gj.at[pl.ds(start, GW)])
            return carry

        lax.fori_loop(0, NWIN, body, 0)

    return k


def _make_scatter():
    mesh = plsc.VectorSubcoreMesh(core_axis_name="c", subcore_axis_name="s")

    @functools.partial(
        pl.kernel,
        mesh=mesh,
        out_type=jax.ShapeDtypeStruct((2, NA, D), _f32),
        scratch_types=(
            pltpu.VMEM((GW,), jnp.int32),
            pltpu.VMEM((GW, 16), _f32),
            pltpu.VMEM((ZR, 16), _f32),
            pltpu.VMEM_SHARED((NA, 16), _f32),
        ),
        compiler_params=pltpu.CompilerParams(use_tc_tiling_on_sc=False),
        name="pair_scatter",
    )
    def k(v, ii, out, idxv, vv, zv, tbl):
        c = lax.axis_index("c")
        s = lax.axis_index("s")
        wid = s * 2 + c
        base = wid * PCHUNK
        row0 = s * RPT

        def zfill(i, carry):
            zv[i, :] = jnp.zeros((16,), _f32)
            return carry

        lax.fori_loop(0, ZR, zfill, 0)

        for ch in range(D // 16):
            co = ch * 16

            def zero_body(i, carry):
                pltpu.sync_copy(zv, tbl.at[pl.ds(row0 + i * ZR, ZR)])
                return carry

            lax.fori_loop(0, RPT // ZR, zero_body, 0)
            plsc.subcore_barrier()

            def win(w, carry):
                start = base + w * GW
                pltpu.sync_copy(ii.at[pl.ds(start, GW)], idxv)
                pltpu.sync_copy(v.at[pl.ds(start, GW), pl.ds(co, 16)], vv)
                pltpu.sync_copy(vv, tbl.at[idxv], add=True)
                return carry

            lax.fori_loop(0, NWIN, win, 0)
            plsc.subcore_barrier()
            pltpu.sync_copy(
                tbl.at[pl.ds(row0, RPT)],
                out.at[c, pl.ds(row0, RPT), pl.ds(co, 16)],
            )

    return k


def _pair_call(first, gi, gj, dist2, diff, ssq, w1, b1, w2, b2, iw1, iw2, gsel):
    cin = NE if first else C
    dg = 16 if first else D

    def body(gi_ref, gj_ref, d_ref, df_ref, sq_ref,
             w1_ref, b1_ref, w2_ref, b2_ref, iw1_ref, iw2_ref, gs_ref, v_ref):
        inv = lax.rsqrt(sq_ref[0, 0])
        nd = df_ref[...] * inv
        x_ = nd[:, 0:1]
        y_ = nd[:, 1:2]
        z_ = nd[:, 2:3]
        x2 = x_ * x_
        y2 = y_ * y_
        z2 = z_ * z_
        third = _f32(1.0 / 3.0)
        dp5 = [
            2.0 * third * x2 - third * y2 - third * z2,
            2.0 * third * y2 - third * x2 - third * z2,
            x_ * y_,
            x_ * z_,
            y_ * z_,
        ]
        fc = 0.5 * (jnp.cos(_f32(np.pi / RC) * d_ref[...]) + 1.0)
        f2 = fc * fc
        f3 = f2 * fc
        f4 = f3 * fc
        basis = jnp.concatenate([fc, f2, f3, f4], axis=1)
        basis_exp = jnp.concatenate([basis] * C, axis=1)

        xcat = jnp.concatenate([gi_ref[:, :cin], gj_ref[:, :cin]], axis=1)
        h = jnp.tanh(jnp.dot(xcat, w1_ref[...], preferred_element_type=_f32)
                     + b1_ref[...])
        h = jnp.tanh(jnp.dot(h, w2_ref[...], preferred_element_type=_f32)
                     + b2_ref[...])
        i1 = jnp.dot(h * basis_exp, gs_ref[...], preferred_element_type=_f32)
        i1 = jnp.tanh(jnp.dot(i1, iw1_ref[...], preferred_element_type=_f32))
        i1 = jnp.tanh(jnp.dot(i1, iw2_ref[...], preferred_element_type=_f32))
        i1_1 = i1[:, 0:16]
        i1_2 = i1[:, 16:32]
        i1_3 = i1[:, 32:48]
        i1_4 = i1[:, 48:64]
        i1_5 = i1[:, 64:80]
        parts = [i1_1]
        for xx in range(3):
            t = nd[:, xx:xx + 1] * i1_2
            if not first:
                t = t + (gi_ref[:, 16 + 16 * xx:32 + 16 * xx]
                         + gj_ref[:, 16 + 16 * xx:32 + 16 * xx]) * i1_4
            parts.append(t)
        for xx in range(5):
            t = dp5[xx] * i1_3
            if not first:
                t = t + (gi_ref[:, 64 + 16 * xx:80 + 16 * xx]
                         + gj_ref[:, 64 + 16 * xx:80 + 16 * xx]) * i1_5
            parts.append(t)
        v_ref[...] = jnp.concatenate(parts, axis=1)

    return pl.pallas_call(
        body,
        grid=(NP // BP,),
        in_specs=[
            pl.BlockSpec((BP, dg), lambda i: (i, 0)),
            pl.BlockSpec((BP, dg), lambda i: (i, 0)),
            pl.BlockSpec((BP, 1), lambda i: (i, 0)),
            pl.BlockSpec((BP, 3), lambda i: (i, 0)),
            pl.BlockSpec((1, 1), lambda i: (0, 0), memory_space=pltpu.SMEM),
            _full_spec(w1.shape),
            _full_spec(b1.shape),
            _full_spec(w2.shape),
            _full_spec(b2.shape),
            _full_spec(iw1.shape),
            _full_spec(iw2.shape),
            _full_spec(gsel.shape),
        ],
        out_specs=pl.BlockSpec((BP, D), lambda i: (i, 0)),
        out_shape=jax.ShapeDtypeStruct((NP, D), _f32),
    )(gi, gj, dist2, diff, ssq, w1, b1, w2, b2, iw1, iw2, gsel)


def _atom_call(first, last, partials, prev, out_prev, ws):
    dprev = 16 if first else D
    n_out = 2 if last else 5

    def body(*refs):
        (pa_ref, pb_ref, prev_ref, op_ref,
         d1wi, d1wj, d2wi, d2wj, pw1, pb1, pw2, pb2, pp3w, pp5w,
         ow1, ob1, ow2, ob2, outw, outb) = refs[:20]
        idx = 20
        if first:
            res1 = refs[idx]
            idx += 1
        if not last:
            n3wi, n3wj, n5wi, n5wj = refs[idx:idx + 4]
            idx += 4
        out_refs = refs[idx:]
        if last:
            (o_ref,) = out_refs
        else:
            (o_ref, p_ref, ti_ref, tj_ref) = out_refs

        pn = pa_ref[0] + pb_ref[0]
        p1n = pn[:, 0:16]
        dot = functools.partial(jnp.dot, preferred_element_type=_f32)
        dot1 = jnp.zeros((pn.shape[0], 16), _f32)
        for xx in range(5):
            sx = pn[:, 64 + 16 * xx:80 + 16 * xx]
            dot1 = dot1 + dot(sx, d1wi[...]) * dot(sx, d1wj[...])
        dot2 = jnp.zeros((pn.shape[0], 16), _f32)
        for xx in range(3):
            sx = pn[:, 16 + 16 * xx:32 + 16 * xx]
            dot2 = dot2 + dot(sx, d2wi[...]) * dot(sx, d2wj[...])
        p1t = jnp.concatenate([dot1, dot2, p1n], axis=1)
        p1t = jnp.tanh(dot(p1t, pw1[...]) + pb1[...])
        p1t = jnp.tanh(dot(p1t, pw2[...]) + pb2[...])
        p1t1 = p1t[:, 0:16]
        p1t2 = p1t[:, 16:32]
        p1t3 = p1t[:, 32:48]
        p3t = [dot(pn[:, 16 + 16 * xx:32 + 16 * xx] * p1t2, pp3w[...])
               for xx in range(3)]
        p5t = [dot(pn[:, 64 + 16 * xx:80 + 16 * xx] * p1t3, pp5w[...])
               for xx in range(5)]
        h = dot(p1t1, ow1[...]) + ob1[...]
        h = dot(h, ow2[...]) + ob2[...]
        o_ref[...] = op_ref[...] + dot(h, outw[...]) + outb[...]

        if first:
            p1 = dot(prev_ref[:, 0:4], res1[...]) + p1t1
            p3x = p3t
            p5x = p5t
        else:
            p1 = prev_ref[:, 0:16] + p1t1
            p3x = [prev_ref[:, 16 + 16 * xx:32 + 16 * xx] + p3t[xx]
                   for xx in range(3)]
            p5x = [prev_ref[:, 64 + 16 * xx:80 + 16 * xx] + p5t[xx]
                   for xx in range(5)]
        if not last:
            p_ref[...] = jnp.concatenate([p1] + p3x + p5x, axis=1)
            ti_ref[...] = jnp.concatenate(
                [p1] + [dot(q, n3wi[...]) for q in p3x]
                + [dot(q, n5wi[...]) for q in p5x], axis=1)
            tj_ref[...] = jnp.concatenate(
                [p1] + [dot(q, n3wj[...]) for q in p3x]
                + [dot(q, n5wj[...]) for q in p5x], axis=1)

    in_arrays = [partials, partials, prev, out_prev] + ws
    in_specs = [
        pl.BlockSpec((1, BA, D), lambda i: (0, i, 0)),
        pl.BlockSpec((1, BA, D), lambda i: (1, i, 0)),
        pl.BlockSpec((BA, dprev), lambda i: (i, 0)),
        pl.BlockSpec((BA, 1), lambda i: (i, 0)),
    ] + [_full_spec(w.shape) for w in ws]
    out_shapes = [jax.ShapeDtypeStruct((NA, 1), _f32)]
    out_specs = [pl.BlockSpec((BA, 1), lambda i: (i, 0))]
    if not last:
        out_shapes += [jax.ShapeDtypeStruct((NA, D), _f32)] * 3
        out_specs += [pl.BlockSpec((BA, D), lambda i: (i, 0))] * 3

    return pl.pallas_call(
        body,
        grid=(NA // BA,),
        in_specs=in_specs,
        out_specs=out_specs,
        out_shape=out_shapes,
    )(*in_arrays)


_GSEL = np.zeros((C * NB, C), np.float32)
for _j in range(C * NB):
    _GSEL[_j, _j // NB] = 1.0


def kernel(ind_1, elems, coord, ind_2, dist, diff, params):
    ii = ind_2[:, 0]
    ij = ind_2[:, 1]
    dist2 = dist[:, None]
    ssq = _ssq(diff)
    t0 = _onehot(elems[:, None].astype(jnp.int32))
    gsel = jnp.asarray(_GSEL)

    gather16 = _make_gather(16)
    gather144 = _make_gather(D)
    scatter = _make_scatter()

    out_acc = jnp.zeros((NA, 1), _f32)
    prev = t0
    ti = t0
    tj = t0
    for b in range(DEPTH):
        bp = params["block%d" % b]
        first = b == 0
        last = b == DEPTH - 1
        gk = gather16 if first else gather144
        gi, gj = gk(ti, tj, ii, ij)
        (w1, b1), (w2, b2) = bp["pi1"]
        iw1, iw2 = bp["ii1"]
        v = _pair_call(first, gi, gj, dist2, diff, ssq,
                       w1, b1[None, :], w2, b2[None, :], iw1, iw2, gsel)
        partials = scatter(v, ii)
        (pw1, pb1), (pw2, pb2) = bp["pp1"]
        (ow1, ob1), (ow2, ob2) = bp["out_ff"]
        ws = [bp["dot1_wi"], bp["dot1_wj"], bp["dot2_wi"], bp["dot2_wj"],
              pw1, pb1[None, :], pw2, pb2[None, :], bp["pp3_W"], bp["pp5_W"],
              ow1, ob1[None, :], ow2, ob2[None, :], bp["out_W"],
              bp["out_b"][None, :]]
        if first:
            ws.append(params["res1_W"])
        if not last:
            nbp = params["block%d" % (b + 1)]
            ws += [nbp["pix3_wi"], nbp["pix3_wj"], nbp["pix5_wi"], nbp["pix5_wj"]]
        outs = _atom_call(first, last, partials, prev, out_acc, ws)
        if last:
            out_acc = outs[0]
        else:
            out_acc, prev, ti, tj = outs[0], outs[1], outs[2], outs[3]
    return out_acc[:, 0]


# speedup vs baseline: 27.5574x; 1.7531x over previous
import functools

import numpy as np
import jax
import jax.numpy as jnp
from jax import lax
from jax.experimental import pallas as pl
from jax.experimental.pallas import tpu as pltpu
import jax.experimental.pallas.tpu_sc as plsc

C = 16
NB = 4
NE = 4
DEPTH = 4
RC = 4.0
NA = 50000
NP = 800000

BP = 4000
BA = 2000
NWORK = 32
PCHUNK = NP // NWORK
GW = 200
NWIN_G = PCHUNK // GW
SW = 1000
NWIN_S = PCHUNK // SW
RPT = NA // 16
ZR = 125

_f32 = jnp.float32


def _full_spec(shape):
    n = len(shape)
    return pl.BlockSpec(shape, lambda i: (0,) * n)


def _sel(shape, pairs):
    m = np.zeros(shape, np.float32)
    for r, c in pairs:
        m[r, c] = 1.0
    return m

_EB = [np.asarray(_sel((1, C * NB), [(0, 4 * c + k) for c in range(C)]))
       for k in range(NB)]
_GS = _sel((C * NB, C), [(4 * c + k, c) for c in range(C) for k in range(NB)])
_S2_48 = _sel((80, 48), [(16 + c, 16 * x + c) for x in range(3) for c in range(C)])
_S4_48 = _sel((80, 48), [(48 + c, 16 * x + c) for x in range(3) for c in range(C)])
_S3_80 = _sel((80, 80), [(32 + c, 16 * x + c) for x in range(5) for c in range(C)])
_S5_80 = _sel((80, 80), [(64 + c, 16 * x + c) for x in range(5) for c in range(C)])
_E3 = _sel((3, 48), [(x, 16 * x + c) for x in range(3) for c in range(C)])
_M2E = np.zeros((3, 80), np.float32)
for _c in range(C):
    _M2E[:, _c] = [2.0 / 3.0, -1.0 / 3.0, -1.0 / 3.0]
    _M2E[:, 16 + _c] = [-1.0 / 3.0, 2.0 / 3.0, -1.0 / 3.0]
_A3 = _sel((3, 3), [(0, 0), (0, 1), (1, 2)])
_B3 = _sel((3, 3), [(1, 0), (2, 1), (2, 2)])
_M3E = _sel((3, 80), [(j, 16 * (2 + j) + c) for j in range(3) for c in range(C)])
_R5 = _sel((80, 16), [(16 * x + c, c) for x in range(5) for c in range(C)])
_R3 = _sel((48, 16), [(16 * x + c, c) for x in range(3) for c in range(C)])
_S2P = _sel((48, 48), [(16 + c, 16 * x + c) for x in range(3) for c in range(C)])
_S3P = _sel((48, 80), [(32 + c, 16 * x + c) for x in range(5) for c in range(C)])


def _ssq(diff):
    bs = 32000

    def body(d_ref, o_ref):
        @pl.when(pl.program_id(0) == 0)
        def _():
            o_ref[0, 0] = 0.0

        d = d_ref[...]
        o_ref[0, 0] += jnp.sum(d * d)

    return pl.pallas_call(
        body,
        grid=(NP // bs,),
        in_specs=[pl.BlockSpec((bs, 3), lambda i: (i, 0))],
        out_specs=pl.BlockSpec((1, 1), lambda i: (0, 0),
                               memory_space=pltpu.SMEM),
        out_shape=jax.ShapeDtypeStruct((1, 1), _f32),
    )(diff)


def _onehot(elems2d):
    def body(e_ref, t_ref):
        e = e_ref[...]
        cols = [(e == t).astype(_f32) for t in (1, 6, 7, 8)]
        cols.append(jnp.zeros((e.shape[0], 12), _f32))
        t_ref[...] = jnp.concatenate(cols, axis=1)

    return pl.pallas_call(
        body,
        grid=(NA // BA,),
        in_specs=[pl.BlockSpec((BA, 1), lambda i: (i, 0))],
        out_specs=pl.BlockSpec((BA, 16), lambda i: (i, 0)),
        out_shape=jax.ShapeDtypeStruct((NA, 16), _f32),
    )(elems2d)


def _make_gather16():
    mesh = plsc.VectorSubcoreMesh(core_axis_name="c", subcore_axis_name="s")

    @functools.partial(
        pl.kernel,
        mesh=mesh,
        out_type=(
            jax.ShapeDtypeStruct((NP, 16), _f32),
            jax.ShapeDtypeStruct((NP, 16), _f32),
        ),
        scratch_types=(
            pltpu.VMEM((GW,), jnp.int32),
            pltpu.VMEM((GW,), jnp.int32),
            pltpu.VMEM((GW, 16), _f32),
            pltpu.VMEM((GW, 16), _f32),
            pltpu.SemaphoreType.DMA,
            pltpu.SemaphoreType.DMA,
        ),
        compiler_params=pltpu.CompilerParams(use_tc_tiling_on_sc=False),
        name="pair_gather16",
    )
    def k(t1, ii, ij, g1i, g1j, iiv, ijv, r1i, r1j, s1, s2):
        wid = lax.axis_index("s") * 2 + lax.axis_index("c")
        base = wid * PCHUNK

        def body(w, carry):
            start = base + w * GW
            pltpu.sync_copy(ii.at[pl.ds(start, GW)], iiv)
            pltpu.sync_copy(ij.at[pl.ds(start, GW)], ijv)
            c1 = pltpu.async_copy(t1.at[iiv], r1i, s1)
            c2 = pltpu.async_copy(t1.at[ijv], r1j, s2)
            c1.wait()
            c2.wait()
            o1 = pltpu.async_copy(r1i, g1i.at[pl.ds(start, GW)], s1)
            o2 = pltpu.async_copy(r1j, g1j.at[pl.ds(start, GW)], s2)
            o1.wait()
            o2.wait()
            return carry

        lax.fori_loop(0, NWIN_G, body, 0)

    return k


def _make_gather6():
    mesh = plsc.VectorSubcoreMesh(core_axis_name="c", subcore_axis_name="s")

    @functools.partial(
        pl.kernel,
        mesh=mesh,
        out_type=(
            jax.ShapeDtypeStruct((NP, 16), _f32),
            jax.ShapeDtypeStruct((NP, 16), _f32),
            jax.ShapeDtypeStruct((NP, 48), _f32),
            jax.ShapeDtypeStruct((NP, 48), _f32),
            jax.ShapeDtypeStruct((NP, 80), _f32),
            jax.ShapeDtypeStruct((NP, 80), _f32),
        ),
        scratch_types=(
            pltpu.VMEM((GW,), jnp.int32),
            pltpu.VMEM((GW,), jnp.int32),
            pltpu.VMEM((GW, 16), _f32),
            pltpu.VMEM((GW, 16), _f32),
            pltpu.VMEM((GW, 48), _f32),
            pltpu.VMEM((GW, 48), _f32),
            pltpu.VMEM((GW, 80), _f32),
            pltpu.VMEM((GW, 80), _f32),
        ) + (pltpu.SemaphoreType.DMA,) * 6,
        compiler_params=pltpu.CompilerParams(use_tc_tiling_on_sc=False),
        name="pair_gather6",
    )
    def k(t1, t3i, t3j, t5i, t5j, ii, ij,
          g1i, g1j, g3i, g3j, g5i, g5j,
          iiv, ijv, r1i, r1j, r3i, r3j, r5i, r5j,
          s1, s2, s3, s4, s5, s6):
        wid = lax.axis_index("s") * 2 + lax.axis_index("c")
        base = wid * PCHUNK

        def body(w, carry):
            start = base + w * GW
            pltpu.sync_copy(ii.at[pl.ds(start, GW)], iiv)
            pltpu.sync_copy(ij.at[pl.ds(start, GW)], ijv)
            cs = [
                pltpu.async_copy(t1.at[iiv], r1i, s1),
                pltpu.async_copy(t1.at[ijv], r1j, s2),
                pltpu.async_copy(t3i.at[iiv], r3i, s3),
                pltpu.async_copy(t3j.at[ijv], r3j, s4),
                pltpu.async_copy(t5i.at[iiv], r5i, s5),
                pltpu.async_copy(t5j.at[ijv], r5j, s6),
            ]
            for cdesc in cs:
                cdesc.wait()
            sl = pl.ds(start, GW)
            os = [
                pltpu.async_copy(r1i, g1i.at[sl], s1),
                pltpu.async_copy(r1j, g1j.at[sl], s2),
                pltpu.async_copy(r3i, g3i.at[sl], s3),
                pltpu.async_copy(r3j, g3j.at[sl], s4),
                pltpu.async_copy(r5i, g5i.at[sl], s5),
                pltpu.async_copy(r5j, g5j.at[sl], s6),
            ]
            for odesc in os:
                odesc.wait()
            return carry

        lax.fori_loop(0, NWIN_G, body, 0)

    return k


def _make_scatter():
    mesh = plsc.VectorSubcoreMesh(core_axis_name="c", subcore_axis_name="s")

    @functools.partial(
        pl.kernel,
        mesh=mesh,
        out_type=(
            jax.ShapeDtypeStruct((2, NA, 16), _f32),
            jax.ShapeDtypeStruct((2, NA, 48), _f32),
            jax.ShapeDtypeStruct((2, NA, 80), _f32),
        ),
        scratch_types=(
            pltpu.VMEM((SW,), jnp.int32),
            pltpu.VMEM((SW, 16), _f32),
            pltpu.VMEM((ZR, 16), _f32),
            pltpu.VMEM_SHARED((NA, 16), _f32),
        ),
        compiler_params=pltpu.CompilerParams(use_tc_tiling_on_sc=False),
        name="pair_scatter",
    )
    def k(v1, v3, v5, ii, o1, o3, o5, idxv, vv, zv, tbl):
        c = lax.axis_index("c")
        s = lax.axis_index("s")
        wid = s * 2 + c
        base = wid * PCHUNK
        row0 = s * RPT

        def zfill(i, carry):
            zv[i, :] = jnp.zeros((16,), _f32)
            return carry

        lax.fori_loop(0, ZR, zfill, 0)

        chunks = ([(v1, o1, 0)]
                  + [(v3, o3, 16 * x) for x in range(3)]
                  + [(v5, o5, 16 * x) for x in range(5)])
        for vref, oref, co in chunks:

            def zero_body(i, carry):
                pltpu.sync_copy(zv, tbl.at[pl.ds(row0 + i * ZR, ZR)])
                return carry

            lax.fori_loop(0, RPT // ZR, zero_body, 0)
            plsc.subcore_barrier()

            def win(w, carry):
                start = base + w * SW
                pltpu.sync_copy(ii.at[pl.ds(start, SW)], idxv)
                pltpu.sync_copy(vref.at[pl.ds(start, SW), pl.ds(co, 16)], vv)
                pltpu.sync_copy(vv, tbl.at[idxv], add=True)
                return carry

            lax.fori_loop(0, NWIN_S, win, 0)
            plsc.subcore_barrier()
            pltpu.sync_copy(
                tbl.at[pl.ds(row0, RPT)],
                oref.at[c, pl.ds(row0, RPT), pl.ds(co, 16)],
            )

    return k


def _pair_pallas(first, g1i, g1j, g35, dist2, diff, ssq, wts):
    names_common = ["w1a", "w1b", "b1", "w2", "b2", "iw1", "iw2",
                    "eb0", "eb1", "eb2", "eb3", "gs",
                    "e3", "m2e", "a3", "b3", "m3e",
                    "s248", "s380"]
    names = names_common + ([] if first else ["s448", "s580"])
    warrs = [wts[n] for n in names]

    def body(*refs):
        if first:
            g1i_ref, g1j_ref, d_ref, df_ref, sq_ref = refs[:5]
            wrefs = refs[5:5 + len(names)]
            v1_ref, v3_ref, v5_ref = refs[5 + len(names):]
            g3i_ref = g3j_ref = g5i_ref = g5j_ref = None
        else:
            (g1i_ref, g1j_ref, g3i_ref, g3j_ref, g5i_ref, g5j_ref,
             d_ref, df_ref, sq_ref) = refs[:9]
            wrefs = refs[9:9 + len(names)]
            v1_ref, v3_ref, v5_ref = refs[9 + len(names):]
        w = dict(zip(names, wrefs))
        dot = functools.partial(jnp.dot, preferred_element_type=_f32)

        inv = lax.rsqrt(sq_ref[0, 0])
        nd = df_ref[...] * inv
        nd2 = nd * nd
        cross = dot(nd, w["a3"][...]) * dot(nd, w["b3"][...])
        dp5e = dot(nd2, w["m2e"][...]) + dot(cross, w["m3e"][...])
        nde = dot(nd, w["e3"][...])

        fc = 0.5 * (jnp.cos(_f32(np.pi / RC) * d_ref[...]) + 1.0)
        t2 = fc * fc
        t3 = t2 * fc
        t4 = t2 * t2
        basis_exp = (dot(fc, w["eb0"][...]) + dot(t2, w["eb1"][...])
                     + dot(t3, w["eb2"][...]) + dot(t4, w["eb3"][...]))

        h = jnp.tanh(dot(g1i_ref[...], w["w1a"][...])
                     + dot(g1j_ref[...], w["w1b"][...]) + w["b1"][...])
        h = jnp.tanh(dot(h, w["w2"][...]) + w["b2"][...])
        i1 = dot(h * basis_exp, w["gs"][...])
        i1 = jnp.tanh(dot(i1, w["iw1"][...]))
        i1 = jnp.tanh(dot(i1, w["iw2"][...]))

        v1_ref[...] = i1[:, 0:16]
        v3 = nde * dot(i1, w["s248"][...])
        v5 = dp5e * dot(i1, w["s380"][...])
        if not first:
            v3 = v3 + (g3i_ref[...] + g3j_ref[...]) * dot(i1, w["s448"][...])
            v5 = v5 + (g5i_ref[...] + g5j_ref[...]) * dot(i1, w["s580"][...])
        v3_ref[...] = v3
        v5_ref[...] = v5

    if first:
        arrays = [g1i, g1j, dist2, diff, ssq] + warrs
        in_specs = [
            pl.BlockSpec((BP, 16), lambda i: (i, 0)),
            pl.BlockSpec((BP, 16), lambda i: (i, 0)),
            pl.BlockSpec((BP, 1), lambda i: (i, 0)),
            pl.BlockSpec((BP, 3), lambda i: (i, 0)),
            pl.BlockSpec((1, 1), lambda i: (0, 0), memory_space=pltpu.SMEM),
        ] + [_full_spec(a.shape) for a in warrs]
    else:
        g3i, g3j, g5i, g5j = g35
        arrays = [g1i, g1j, g3i, g3j, g5i, g5j, dist2, diff, ssq] + warrs
        in_specs = [
            pl.BlockSpec((BP, 16), lambda i: (i, 0)),
            pl.BlockSpec((BP, 16), lambda i: (i, 0)),
            pl.BlockSpec((BP, 48), lambda i: (i, 0)),
            pl.BlockSpec((BP, 48), lambda i: (i, 0)),
            pl.BlockSpec((BP, 80), lambda i: (i, 0)),
            pl.BlockSpec((BP, 80), lambda i: (i, 0)),
            pl.BlockSpec((BP, 1), lambda i: (i, 0)),
            pl.BlockSpec((BP, 3), lambda i: (i, 0)),
            pl.BlockSpec((1, 1), lambda i: (0, 0), memory_space=pltpu.SMEM),
        ] + [_full_spec(a.shape) for a in warrs]

    return pl.pallas_call(
        body,
        grid=(NP // BP,),
        in_specs=in_specs,
        out_specs=[
            pl.BlockSpec((BP, 16), lambda i: (i, 0)),
            pl.BlockSpec((BP, 48), lambda i: (i, 0)),
            pl.BlockSpec((BP, 80), lambda i: (i, 0)),
        ],
        out_shape=[
            jax.ShapeDtypeStruct((NP, 16), _f32),
            jax.ShapeDtypeStruct((NP, 48), _f32),
            jax.ShapeDtypeStruct((NP, 80), _f32),
        ],
    )(*arrays)


def _atom_pallas(first, last, o1, o3, o5, prevs, out_prev, wts):
    names = ["d1i", "d1j", "r5", "d2i", "d2j", "r3",
             "pw1a", "pw1b", "pw1c", "pb1", "pw2", "pb2",
             "s2p", "s3p", "p3b", "p5b",
             "ow1", "ob1", "ow2", "ob2", "outw", "outb"]
    if first:
        names.append("res1p")
    if not last:
        names += ["x3i", "x3j", "x5i", "x5j"]
    warrs = [wts[n] for n in names]
    nin_prev = 1 if first else 3

    def body(*refs):
        pa1, pb1_, pa3, pb3, pa5, pb5 = refs[:6]
        prefs = refs[6:6 + nin_prev]
        op_ref = refs[6 + nin_prev]
        wrefs = refs[7 + nin_prev:7 + nin_prev + len(names)]
        out_refs = refs[7 + nin_prev + len(names):]
        w = dict(zip(names, wrefs))
        dot = functools.partial(jnp.dot, preferred_element_type=_f32)

        p1n = pa1[0] + pb1_[0]
        p3n = pa3[0] + pb3[0]
        p5n = pa5[0] + pb5[0]

        a5 = dot(p5n, w["d1i"][...])
        b5 = dot(p5n, w["d1j"][...])
        dot1 = dot(a5 * b5, w["r5"][...])
        a3 = dot(p3n, w["d2i"][...])
        b3 = dot(p3n, w["d2j"][...])
        dot2 = dot(a3 * b3, w["r3"][...])

        p1t = jnp.tanh(dot(dot1, w["pw1a"][...]) + dot(dot2, w["pw1b"][...])
                       + dot(p1n, w["pw1c"][...]) + w["pb1"][...])
        p1t = jnp.tanh(dot(p1t, w["pw2"][...]) + w["pb2"][...])
        p1t1 = p1t[:, 0:16]
        p3t = dot(p3n * dot(p1t, w["s2p"][...]), w["p3b"][...])
        p5t = dot(p5n * dot(p1t, w["s3p"][...]), w["p5b"][...])

        h = dot(p1t1, w["ow1"][...]) + w["ob1"][...]
        h = dot(h, w["ow2"][...]) + w["ob2"][...]
        o_ref = out_refs[0]
        o_ref[...] = op_ref[...] + dot(h, w["outw"][...]) + w["outb"][...]

        if first:
            p1 = dot(prefs[0][...], w["res1p"][...]) + p1t1
            p3 = p3t
            p5 = p5t
        else:
            p1 = prefs[0][...] + p1t1
            p3 = prefs[1][...] + p3t
            p5 = prefs[2][...] + p5t
        if not last:
            (p1_ref, p3_ref, p5_ref, t3i_ref, t3j_ref,
             t5i_ref, t5j_ref) = out_refs[1:]
            p1_ref[...] = p1
            p3_ref[...] = p3
            p5_ref[...] = p5
            t3i_ref[...] = dot(p3, w["x3i"][...])
            t3j_ref[...] = dot(p3, w["x3j"][...])
            t5i_ref[...] = dot(p5, w["x5i"][...])
            t5j_ref[...] = dot(p5, w["x5j"][...])

    arrays = [o1, o1, o3, o3, o5, o5] + list(prevs) + [out_prev] + warrs
    in_specs = [
        pl.BlockSpec((1, BA, 16), lambda i: (0, i, 0)),
        pl.BlockSpec((1, BA, 16), lambda i: (1, i, 0)),
        pl.BlockSpec((1, BA, 48), lambda i: (0, i, 0)),
        pl.BlockSpec((1, BA, 48), lambda i: (1, i, 0)),
        pl.BlockSpec((1, BA, 80), lambda i: (0, i, 0)),
        pl.BlockSpec((1, BA, 80), lambda i: (1, i, 0)),
    ]
    for p in prevs:
        in_specs.append(pl.BlockSpec((BA, p.shape[1]), lambda i: (i, 0)))
    in_specs.append(pl.BlockSpec((BA, 1), lambda i: (i, 0)))
    in_specs += [_full_spec(a.shape) for a in warrs]

    out_shapes = [jax.ShapeDtypeStruct((NA, 1), _f32)]
    out_specs = [pl.BlockSpec((BA, 1), lambda i: (i, 0))]
    if not last:
        for cols in (16, 48, 80, 48, 48, 80, 80):
            out_shapes.append(jax.ShapeDtypeStruct((NA, cols), _f32))
            out_specs.append(pl.BlockSpec((BA, cols), lambda i: (i, 0)))

    return pl.pallas_call(
        body,
        grid=(NA // BA,),
        in_specs=in_specs,
        out_specs=out_specs,
        out_shape=out_shapes,
    )(*arrays)


def kernel(ind_1, elems, coord, ind_2, dist, diff, params):
    ii = ind_2[:, 0]
    ij = ind_2[:, 1]
    dist2 = dist[:, None]
    ssq = _ssq(diff)
    t0 = _onehot(elems[:, None].astype(jnp.int32))

    consts = {
        "eb0": _EB[0], "eb1": _EB[1], "eb2": _EB[2], "eb3": _EB[3],
        "gs": _GS, "e3": _E3, "m2e": _M2E, "a3": _A3, "b3": _B3,
        "m3e": _M3E, "s248": _S2_48, "s380": _S3_80,
        "s448": _S4_48, "s580": _S5_80,
    }
    consts = {k: jnp.asarray(v) for k, v in consts.items()}
    aconsts = {"r5": jnp.asarray(_R5), "r3": jnp.asarray(_R3),
               "s2p": jnp.asarray(_S2P), "s3p": jnp.asarray(_S3P)}
    eye3 = jnp.eye(3, dtype=_f32)
    eye5 = jnp.eye(5, dtype=_f32)

    gather16 = _make_gather16()
    gather6 = _make_gather6()
    scatter = _make_scatter()

    out_acc = jnp.zeros((NA, 1), _f32)
    prevs = (t0,)
    t1 = t0
    t3i = t3j = t5i = t5j = None
    for b in range(DEPTH):
        bp = params["block%d" % b]
        first = b == 0
        last = b == DEPTH - 1
        if first:
            g1i, g1j = gather16(t1, ii, ij)
            g35 = None
        else:
            g1i, g1j, g3i, g3j, g5i, g5j = gather6(
                t1, t3i, t3j, t5i, t5j, ii, ij)
            g35 = (g3i, g3j, g5i, g5j)
        (w1, b1), (w2, b2) = bp["pi1"]
        iw1, iw2 = bp["ii1"]
        cin = NE if first else C
        w1a = w1[:cin]
        w1b = w1[cin:]
        if first:
            w1a = jnp.zeros((16, C), _f32).at[:cin].set(w1a)
            w1b = jnp.zeros((16, C), _f32).at[:cin].set(w1b)
        wts = dict(consts)
        wts.update(w1a=w1a, w1b=w1b, b1=b1[None, :], w2=w2, b2=b2[None, :],
                   iw1=iw1, iw2=iw2)
        v1, v3, v5 = _pair_pallas(first, g1i, g1j, g35, dist2, diff, ssq, wts)
        o1, o3, o5 = scatter(v1, v3, v5, ii)

        (pw1, pb1), (pw2, pb2) = bp["pp1"]
        (ow1, ob1), (ow2, ob2) = bp["out_ff"]
        awts = dict(aconsts)
        awts.update(
            d1i=jnp.kron(eye5, bp["dot1_wi"]), d1j=jnp.kron(eye5, bp["dot1_wj"]),
            d2i=jnp.kron(eye3, bp["dot2_wi"]), d2j=jnp.kron(eye3, bp["dot2_wj"]),
            pw1a=pw1[0:16], pw1b=pw1[16:32], pw1c=pw1[32:48],
            pb1=pb1[None, :], pw2=pw2, pb2=pb2[None, :],
            p3b=jnp.kron(eye3, bp["pp3_W"]), p5b=jnp.kron(eye5, bp["pp5_W"]),
            ow1=ow1, ob1=ob1[None, :], ow2=ow2, ob2=ob2[None, :],
            outw=bp["out_W"], outb=bp["out_b"][None, :])
        if first:
            awts["res1p"] = jnp.zeros((16, C), _f32).at[:NE].set(
                params["res1_W"])
        if not last:
            nbp = params["block%d" % (b + 1)]
            awts.update(
                x3i=jnp.kron(eye3, nbp["pix3_wi"]),
                x3j=jnp.kron(eye3, nbp["pix3_wj"]),
                x5i=jnp.kron(eye5, nbp["pix5_wi"]),
                x5j=jnp.kron(eye5, nbp["pix5_wj"]))
        outs = _atom_pallas(first, last, o1, o3, o5, prevs, out_acc, awts)
        if last:
            out_acc = outs[0]
        else:
            out_acc = outs[0]
            prevs = (outs[1], outs[2], outs[3])
            t1, t3i, t3j, t5i, t5j = (outs[1], outs[4], outs[5],
                                      outs[6], outs[7])
    return out_acc[:, 0]


# speedup vs baseline: 29.6117x; 1.0745x over previous
import functools

import numpy as np
import jax
import jax.numpy as jnp
from jax import lax
from jax.experimental import pallas as pl
from jax.experimental.pallas import tpu as pltpu
import jax.experimental.pallas.tpu_sc as plsc

C = 16
NB = 4
NE = 4
DEPTH = 4
RC = 4.0
NA = 50000
NP = 800000

BP = 4000
BA = 2000
NWORK = 32
PCHUNK = NP // NWORK
GW = 200
NWIN_G = PCHUNK // GW
SW = 1000
NWIN_S = PCHUNK // SW
RPT = NA // 16
ZR = 125

_f32 = jnp.float32


def _full_spec(shape):
    n = len(shape)
    return pl.BlockSpec(shape, lambda i: (0,) * n)


def _sel(shape, pairs):
    m = np.zeros(shape, np.float32)
    for r, c in pairs:
        m[r, c] = 1.0
    return m

_EB = [np.asarray(_sel((1, C * NB), [(0, 4 * c + k) for c in range(C)]))
       for k in range(NB)]
_GS = _sel((C * NB, C), [(4 * c + k, c) for c in range(C) for k in range(NB)])
_S2_48 = _sel((80, 48), [(16 + c, 16 * x + c) for x in range(3) for c in range(C)])
_S4_48 = _sel((80, 48), [(48 + c, 16 * x + c) for x in range(3) for c in range(C)])
_S3_80 = _sel((80, 80), [(32 + c, 16 * x + c) for x in range(5) for c in range(C)])
_S5_80 = _sel((80, 80), [(64 + c, 16 * x + c) for x in range(5) for c in range(C)])
_E3 = _sel((3, 48), [(x, 16 * x + c) for x in range(3) for c in range(C)])
_M2E = np.zeros((3, 80), np.float32)
for _c in range(C):
    _M2E[:, _c] = [2.0 / 3.0, -1.0 / 3.0, -1.0 / 3.0]
    _M2E[:, 16 + _c] = [-1.0 / 3.0, 2.0 / 3.0, -1.0 / 3.0]
_A3 = _sel((3, 3), [(0, 0), (0, 1), (1, 2)])
_B3 = _sel((3, 3), [(1, 0), (2, 1), (2, 2)])
_M3E = _sel((3, 80), [(j, 16 * (2 + j) + c) for j in range(3) for c in range(C)])
_R5 = _sel((80, 16), [(16 * x + c, c) for x in range(5) for c in range(C)])
_R3 = _sel((48, 16), [(16 * x + c, c) for x in range(3) for c in range(C)])
_S2P = _sel((48, 48), [(16 + c, 16 * x + c) for x in range(3) for c in range(C)])
_S3P = _sel((48, 80), [(32 + c, 16 * x + c) for x in range(5) for c in range(C)])
_SEL3 = _sel((128, 48), [(c, c) for c in range(48)])
_SEL5 = _sel((128, 80), [(48 + c, c) for c in range(80)])


def _ssq(diff):
    bs = 32000

    def body(d_ref, o_ref):
        @pl.when(pl.program_id(0) == 0)
        def _():
            o_ref[0, 0] = 0.0

        d = d_ref[...]
        o_ref[0, 0] += jnp.sum(d * d)

    return pl.pallas_call(
        body,
        grid=(NP // bs,),
        in_specs=[pl.BlockSpec((bs, 3), lambda i: (i, 0))],
        out_specs=pl.BlockSpec((1, 1), lambda i: (0, 0),
                               memory_space=pltpu.SMEM),
        out_shape=jax.ShapeDtypeStruct((1, 1), _f32),
    )(diff)


def _onehot(elems2d, w1a, w1b):
    def body(e_ref, wa_ref, wb_ref, t_ref, u_ref, v_ref):
        e = e_ref[...]
        cols = [(e == t).astype(_f32) for t in (1, 6, 7, 8)]
        cols.append(jnp.zeros((e.shape[0], 12), _f32))
        t = jnp.concatenate(cols, axis=1)
        t_ref[...] = t
        u_ref[...] = jnp.dot(t, wa_ref[...], preferred_element_type=_f32)
        v_ref[...] = jnp.dot(t, wb_ref[...], preferred_element_type=_f32)

    return pl.pallas_call(
        body,
        grid=(NA // BA,),
        in_specs=[pl.BlockSpec((BA, 1), lambda i: (i, 0)),
                  _full_spec(w1a.shape), _full_spec(w1b.shape)],
        out_specs=[pl.BlockSpec((BA, 16), lambda i: (i, 0))] * 3,
        out_shape=[jax.ShapeDtypeStruct((NA, 16), _f32)] * 3,
    )(elems2d, w1a, w1b)


def _make_gatherx16():
    mesh = plsc.VectorSubcoreMesh(core_axis_name="c", subcore_axis_name="s")

    @functools.partial(
        pl.kernel,
        mesh=mesh,
        out_type=jax.ShapeDtypeStruct((NP, 16), _f32),
        scratch_types=(
            pltpu.VMEM((GW,), jnp.int32),
            pltpu.VMEM((GW,), jnp.int32),
            pltpu.VMEM((GW, 16), _f32),
            pltpu.VMEM((GW, 16), _f32),
            pltpu.SemaphoreType.DMA,
            pltpu.SemaphoreType.DMA,
        ),
        compiler_params=pltpu.CompilerParams(use_tc_tiling_on_sc=False),
        name="pair_gatherx16",
    )
    def k(u, v, ii, ij, xw, iiv, ijv, r1i, r1j, s1, s2):
        wid = lax.axis_index("s") * 2 + lax.axis_index("c")
        base = wid * PCHUNK

        def body(w, carry):
            start = base + w * GW
            pltpu.sync_copy(ii.at[pl.ds(start, GW)], iiv)
            pltpu.sync_copy(ij.at[pl.ds(start, GW)], ijv)
            c1 = pltpu.async_copy(u.at[iiv], r1i, s1)
            c2 = pltpu.async_copy(v.at[ijv], r1j, s2)
            c1.wait()
            c2.wait()

            def addrow(r, carry2):
                r1i[r, :] = r1i[r, :] + r1j[r, :]
                return carry2

            lax.fori_loop(0, GW, addrow, 0)
            o1 = pltpu.async_copy(r1i, xw.at[pl.ds(start, GW)], s1)
            o1.wait()
            return carry

        lax.fori_loop(0, NWIN_G, body, 0)

    return k


def _make_gather3():
    mesh = plsc.VectorSubcoreMesh(core_axis_name="c", subcore_axis_name="s")

    @functools.partial(
        pl.kernel,
        mesh=mesh,
        out_type=(
            jax.ShapeDtypeStruct((NP, 16), _f32),
            jax.ShapeDtypeStruct((NP, 128), _f32),
        ),
        scratch_types=(
            pltpu.VMEM((GW,), jnp.int32),
            pltpu.VMEM((GW,), jnp.int32),
            pltpu.VMEM((GW, 16), _f32),
            pltpu.VMEM((GW, 16), _f32),
            pltpu.VMEM((GW, 128), _f32),
            pltpu.VMEM((GW, 128), _f32),
        ) + (pltpu.SemaphoreType.DMA,) * 4,
        compiler_params=pltpu.CompilerParams(use_tc_tiling_on_sc=False),
        name="pair_gather3",
    )
    def k(u, v, t35i, t35j, ii, ij, xw, g35,
          iiv, ijv, r1i, r1j, r35i, r35j, s1, s2, s3, s4):
        wid = lax.axis_index("s") * 2 + lax.axis_index("c")
        base = wid * PCHUNK

        def body(w, carry):
            start = base + w * GW
            pltpu.sync_copy(ii.at[pl.ds(start, GW)], iiv)
            pltpu.sync_copy(ij.at[pl.ds(start, GW)], ijv)
            cs = [
                pltpu.async_copy(u.at[iiv], r1i, s1),
                pltpu.async_copy(v.at[ijv], r1j, s2),
                pltpu.async_copy(t35i.at[iiv], r35i, s3),
                pltpu.async_copy(t35j.at[ijv], r35j, s4),
            ]
            for cdesc in cs:
                cdesc.wait()

            def addrow(r, carry2):
                r1i[r, :] = r1i[r, :] + r1j[r, :]
                for cc in range(8):
                    sl = pl.ds(cc * 16, 16)
                    r35i[r, sl] = r35i[r, sl] + r35j[r, sl]
                return carry2

            lax.fori_loop(0, GW, addrow, 0)
            sl = pl.ds(start, GW)
            o1 = pltpu.async_copy(r1i, xw.at[sl], s1)
            o2 = pltpu.async_copy(r35i, g35.at[sl], s3)
            o1.wait()
            o2.wait()
            return carry

        lax.fori_loop(0, NWIN_G, body, 0)

    return k


def _make_scatter():
    mesh = plsc.VectorSubcoreMesh(core_axis_name="c", subcore_axis_name="s")

    @functools.partial(
        pl.kernel,
        mesh=mesh,
        out_type=(
            jax.ShapeDtypeStruct((2, NA, 16), _f32),
            jax.ShapeDtypeStruct((2, NA, 48), _f32),
            jax.ShapeDtypeStruct((2, NA, 80), _f32),
        ),
        scratch_types=(
            pltpu.VMEM((SW,), jnp.int32),
            pltpu.VMEM((SW, 16), _f32),
            pltpu.VMEM((ZR, 16), _f32),
            pltpu.VMEM_SHARED((NA, 16), _f32),
        ),
        compiler_params=pltpu.CompilerParams(use_tc_tiling_on_sc=False),
        name="pair_scatter",
    )
    def k(v1, v3, v5, ii, o1, o3, o5, idxv, vv, zv, tbl):
        c = lax.axis_index("c")
        s = lax.axis_index("s")
        wid = s * 2 + c
        base = wid * PCHUNK
        row0 = s * RPT

        def zfill(i, carry):
            zv[i, :] = jnp.zeros((16,), _f32)
            return carry

        lax.fori_loop(0, ZR, zfill, 0)

        chunks = ([(v1, o1, 0)]
                  + [(v3, o3, 16 * x) for x in range(3)]
                  + [(v5, o5, 16 * x) for x in range(5)])
        for vref, oref, co in chunks:

            def zero_body(i, carry):
                pltpu.sync_copy(zv, tbl.at[pl.ds(row0 + i * ZR, ZR)])
                return carry

            lax.fori_loop(0, RPT // ZR, zero_body, 0)
            plsc.subcore_barrier()

            def win(w, carry):
                start = base + w * SW
                pltpu.sync_copy(ii.at[pl.ds(start, SW)], idxv)
                pltpu.sync_copy(vref.at[pl.ds(start, SW), pl.ds(co, 16)], vv)
                pltpu.sync_copy(vv, tbl.at[idxv], add=True)
                return carry

            lax.fori_loop(0, NWIN_S, win, 0)
            plsc.subcore_barrier()
            pltpu.sync_copy(
                tbl.at[pl.ds(row0, RPT)],
                oref.at[c, pl.ds(row0, RPT), pl.ds(co, 16)],
            )

    return k


def _pair_pallas(first, xw, g35, dist2, diff, ssq, wts):
    names_common = ["b1", "w2", "b2", "iw1", "iw2",
                    "eb0", "eb1", "eb2", "eb3", "gs",
                    "e3", "m2e", "a3", "b3", "m3e",
                    "s248", "s380"]
    names = names_common + ([] if first else ["s448", "s580", "sel3", "sel5"])
    warrs = [wts[n] for n in names]

    def body(*refs):
        if first:
            xw_ref, d_ref, df_ref, sq_ref = refs[:4]
            wrefs = refs[4:4 + len(names)]
            v1_ref, v3_ref, v5_ref = refs[4 + len(names):]
            g35_ref = None
        else:
            xw_ref, g35_ref, d_ref, df_ref, sq_ref = refs[:5]
            wrefs = refs[5:5 + len(names)]
            v1_ref, v3_ref, v5_ref = refs[5 + len(names):]
        w = dict(zip(names, wrefs))
        dot = functools.partial(jnp.dot, preferred_element_type=_f32)

        inv = lax.rsqrt(sq_ref[0, 0])
        nd = df_ref[...] * inv
        nd2 = nd * nd
        cross = dot(nd, w["a3"][...]) * dot(nd, w["b3"][...])
        dp5e = dot(nd2, w["m2e"][...]) + dot(cross, w["m3e"][...])
        nde = dot(nd, w["e3"][...])

        fc = 0.5 * (jnp.cos(_f32(np.pi / RC) * d_ref[...]) + 1.0)
        t2 = fc * fc
        t3 = t2 * fc
        t4 = t2 * t2
        basis_exp = (dot(fc, w["eb0"][...]) + dot(t2, w["eb1"][...])
                     + dot(t3, w["eb2"][...]) + dot(t4, w["eb3"][...]))

        h = jnp.tanh(xw_ref[...] + w["b1"][...])
        h = jnp.tanh(dot(h, w["w2"][...]) + w["b2"][...])
        i1 = dot(h * basis_exp, w["gs"][...])
        i1 = jnp.tanh(dot(i1, w["iw1"][...]))
        i1 = jnp.tanh(dot(i1, w["iw2"][...]))

        v1_ref[...] = i1[:, 0:16]
        v3 = nde * dot(i1, w["s248"][...])
        v5 = dp5e * dot(i1, w["s380"][...])
        if not first:
            g = g35_ref[...]
            v3 = v3 + dot(g, w["sel3"][...]) * dot(i1, w["s448"][...])
            v5 = v5 + dot(g, w["sel5"][...]) * dot(i1, w["s580"][...])
        v3_ref[...] = v3
        v5_ref[...] = v5

    if first:
        arrays = [xw, dist2, diff, ssq] + warrs
        in_specs = [
            pl.BlockSpec((BP, 16), lambda i: (i, 0)),
            pl.BlockSpec((BP, 1), lambda i: (i, 0)),
            pl.BlockSpec((BP, 3), lambda i: (i, 0)),
            pl.BlockSpec((1, 1), lambda i: (0, 0), memory_space=pltpu.SMEM),
        ] + [_full_spec(a.shape) for a in warrs]
    else:
        arrays = [xw, g35, dist2, diff, ssq] + warrs
        in_specs = [
            pl.BlockSpec((BP, 16), lambda i: (i, 0)),
            pl.BlockSpec((BP, 128), lambda i: (i, 0)),
            pl.BlockSpec((BP, 1), lambda i: (i, 0)),
            pl.BlockSpec((BP, 3), lambda i: (i, 0)),
            pl.BlockSpec((1, 1), lambda i: (0, 0), memory_space=pltpu.SMEM),
        ] + [_full_spec(a.shape) for a in warrs]

    return pl.pallas_call(
        body,
        grid=(NP // BP,),
        in_specs=in_specs,
        out_specs=[
            pl.BlockSpec((BP, 16), lambda i: (i, 0)),
            pl.BlockSpec((BP, 48), lambda i: (i, 0)),
            pl.BlockSpec((BP, 80), lambda i: (i, 0)),
        ],
        out_shape=[
            jax.ShapeDtypeStruct((NP, 16), _f32),
            jax.ShapeDtypeStruct((NP, 48), _f32),
            jax.ShapeDtypeStruct((NP, 80), _f32),
        ],
    )(*arrays)


def _atom_pallas(first, last, o1, o3, o5, prevs, out_prev, wts):
    names = ["d1i", "d1j", "r5", "d2i", "d2j", "r3",
             "pw1a", "pw1b", "pw1c", "pb1", "pw2", "pb2",
             "s2p", "s3p", "p3b", "p5b",
             "ow1", "ob1", "ow2", "ob2", "outw", "outb"]
    if first:
        names.append("res1p")
    if not last:
        names += ["x3i", "x3j", "x5i", "x5j", "nw1a", "nw1b"]
    warrs = [wts[n] for n in names]
    nin_prev = 1 if first else 3

    def body(*refs):
        pa1, pb1_, pa3, pb3, pa5, pb5 = refs[:6]
        prefs = refs[6:6 + nin_prev]
        op_ref = refs[6 + nin_prev]
        wrefs = refs[7 + nin_prev:7 + nin_prev + len(names)]
        out_refs = refs[7 + nin_prev + len(names):]
        w = dict(zip(names, wrefs))
        dot = functools.partial(jnp.dot, preferred_element_type=_f32)

        p1n = pa1[0] + pb1_[0]
        p3n = pa3[0] + pb3[0]
        p5n = pa5[0] + pb5[0]

        a5 = dot(p5n, w["d1i"][...])
        b5 = dot(p5n, w["d1j"][...])
        dot1 = dot(a5 * b5, w["r5"][...])
        a3 = dot(p3n, w["d2i"][...])
        b3 = dot(p3n, w["d2j"][...])
        dot2 = dot(a3 * b3, w["r3"][...])

        p1t = jnp.tanh(dot(dot1, w["pw1a"][...]) + dot(dot2, w["pw1b"][...])
                       + dot(p1n, w["pw1c"][...]) + w["pb1"][...])
        p1t = jnp.tanh(dot(p1t, w["pw2"][...]) + w["pb2"][...])
        p1t1 = p1t[:, 0:16]
        p3t = dot(p3n * dot(p1t, w["s2p"][...]), w["p3b"][...])
        p5t = dot(p5n * dot(p1t, w["s3p"][...]), w["p5b"][...])

        h = dot(p1t1, w["ow1"][...]) + w["ob1"][...]
        h = dot(h, w["ow2"][...]) + w["ob2"][...]
        o_ref = out_refs[0]
        o_ref[...] = op_ref[...] + dot(h, w["outw"][...]) + w["outb"][...]

        if first:
            p1 = dot(prefs[0][...], w["res1p"][...]) + p1t1
            p3 = p3t
            p5 = p5t
        else:
            p1 = prefs[0][...] + p1t1
            p3 = prefs[1][...] + p3t
            p5 = prefs[2][...] + p5t
        if not last:
            (p1_ref, p3_ref, p5_ref, u_ref, v_ref,
             t35i_ref, t35j_ref) = out_refs[1:]
            p1_ref[...] = p1
            p3_ref[...] = p3
            p5_ref[...] = p5
            u_ref[...] = dot(p1, w["nw1a"][...])
            v_ref[...] = dot(p1, w["nw1b"][...])
            t35i_ref[...] = jnp.concatenate(
                [dot(p3, w["x3i"][...]), dot(p5, w["x5i"][...])], axis=1)
            t35j_ref[...] = jnp.concatenate(
                [dot(p3, w["x3j"][...]), dot(p5, w["x5j"][...])], axis=1)

    arrays = [o1, o1, o3, o3, o5, o5] + list(prevs) + [out_prev] + warrs
    in_specs = [
        pl.BlockSpec((1, BA, 16), lambda i: (0, i, 0)),
        pl.BlockSpec((1, BA, 16), lambda i: (1, i, 0)),
        pl.BlockSpec((1, BA, 48), lambda i: (0, i, 0)),
        pl.BlockSpec((1, BA, 48), lambda i: (1, i, 0)),
        pl.BlockSpec((1, BA, 80), lambda i: (0, i, 0)),
        pl.BlockSpec((1, BA, 80), lambda i: (1, i, 0)),
    ]
    for p in prevs:
        in_specs.append(pl.BlockSpec((BA, p.shape[1]), lambda i: (i, 0)))
    in_specs.append(pl.BlockSpec((BA, 1), lambda i: (i, 0)))
    in_specs += [_full_spec(a.shape) for a in warrs]

    out_shapes = [jax.ShapeDtypeStruct((NA, 1), _f32)]
    out_specs = [pl.BlockSpec((BA, 1), lambda i: (i, 0))]
    if not last:
        for cols in (16, 48, 80, 16, 16, 128, 128):
            out_shapes.append(jax.ShapeDtypeStruct((NA, cols), _f32))
            out_specs.append(pl.BlockSpec((BA, cols), lambda i: (i, 0)))

    return pl.pallas_call(
        body,
        grid=(NA // BA,),
        in_specs=in_specs,
        out_specs=out_specs,
        out_shape=out_shapes,
    )(*arrays)


def _w1split(params, b):
    bp = params["block%d" % b]
    w1 = bp["pi1"][0][0]
    cin = NE if b == 0 else C
    w1a = w1[:cin]
    w1b = w1[cin:]
    if b == 0:
        w1a = jnp.zeros((16, C), _f32).at[:cin].set(w1a)
        w1b = jnp.zeros((16, C), _f32).at[:cin].set(w1b)
    return w1a, w1b


def kernel(ind_1, elems, coord, ind_2, dist, diff, params):
    ii = ind_2[:, 0]
    ij = ind_2[:, 1]
    dist2 = dist[:, None]
    ssq = _ssq(diff)
    w1a0, w1b0 = _w1split(params, 0)
    t0, u, v = _onehot(elems[:, None].astype(jnp.int32), w1a0, w1b0)

    consts = {
        "eb0": _EB[0], "eb1": _EB[1], "eb2": _EB[2], "eb3": _EB[3],
        "gs": _GS, "e3": _E3, "m2e": _M2E, "a3": _A3, "b3": _B3,
        "m3e": _M3E, "s248": _S2_48, "s380": _S3_80,
        "s448": _S4_48, "s580": _S5_80,
        "sel3": _SEL3, "sel5": _SEL5,
    }
    consts = {k: jnp.asarray(v) for k, v in consts.items()}
    aconsts = {"r5": jnp.asarray(_R5), "r3": jnp.asarray(_R3),
               "s2p": jnp.asarray(_S2P), "s3p": jnp.asarray(_S3P)}
    eye3 = jnp.eye(3, dtype=_f32)
    eye5 = jnp.eye(5, dtype=_f32)

    gatherx16 = _make_gatherx16()
    gather3 = _make_gather3()
    scatter = _make_scatter()

    out_acc = jnp.zeros((NA, 1), _f32)
    prevs = (t0,)
    t35i = t35j = None
    for b in range(DEPTH):
        bp = params["block%d" % b]
        first = b == 0
        last = b == DEPTH - 1
        if first:
            xw = gatherx16(u, v, ii, ij)
            g35 = None
        else:
            xw, g35 = gather3(u, v, t35i, t35j, ii, ij)
        (w1, b1), (w2, b2) = bp["pi1"]
        iw1, iw2 = bp["ii1"]
        wts = dict(consts)
        wts.update(b1=b1[None, :], w2=w2, b2=b2[None, :],
                   iw1=iw1, iw2=iw2)
        v1, v3, v5 = _pair_pallas(first, xw, g35, dist2, diff, ssq, wts)
        o1, o3, o5 = scatter(v1, v3, v5, ii)

        (pw1, pb1), (pw2, pb2) = bp["pp1"]
        (ow1, ob1), (ow2, ob2) = bp["out_ff"]
        awts = dict(aconsts)
        awts.update(
            d1i=jnp.kron(eye5, bp["dot1_wi"]), d1j=jnp.kron(eye5, bp["dot1_wj"]),
            d2i=jnp.kron(eye3, bp["dot2_wi"]), d2j=jnp.kron(eye3, bp["dot2_wj"]),
            pw1a=pw1[0:16], pw1b=pw1[16:32], pw1c=pw1[32:48],
            pb1=pb1[None, :], pw2=pw2, pb2=pb2[None, :],
            p3b=jnp.kron(eye3, bp["pp3_W"]), p5b=jnp.kron(eye5, bp["pp5_W"]),
            ow1=ow1, ob1=ob1[None, :], ow2=ow2, ob2=ob2[None, :],
            outw=bp["out_W"], outb=bp["out_b"][None, :])
        if first:
            awts["res1p"] = jnp.zeros((16, C), _f32).at[:NE].set(
                params["res1_W"])
        if not last:
            nbp = params["block%d" % (b + 1)]
            nw1a, nw1b = _w1split(params, b + 1)
            awts.update(
                x3i=jnp.kron(eye3, nbp["pix3_wi"]),
                x3j=jnp.kron(eye3, nbp["pix3_wj"]),
                x5i=jnp.kron(eye5, nbp["pix5_wi"]),
                x5j=jnp.kron(eye5, nbp["pix5_wj"]),
                nw1a=nw1a, nw1b=nw1b)
        outs = _atom_pallas(first, last, o1, o3, o5, prevs, out_acc, awts)
        if last:
            out_acc = outs[0]
        else:
            out_acc = outs[0]
            prevs = (outs[1], outs[2], outs[3])
            u, v, t35i, t35j = outs[4], outs[5], outs[6], outs[7]
    return out_acc[:, 0]


# speedup vs baseline: 37.0630x; 1.2516x over previous
import functools

import numpy as np
import jax
import jax.numpy as jnp
from jax import lax
from jax.experimental import pallas as pl
from jax.experimental.pallas import tpu as pltpu
import jax.experimental.pallas.tpu_sc as plsc

C = 16
NB = 4
NE = 4
DEPTH = 4
RC = 4.0
NA = 50000
NP = 800000

BP = 4000
BA = 2000
NWORK = 32
PCHUNK = NP // NWORK
GW = 200
NWIN_G = PCHUNK // GW
SW = 1000
NWIN_S = PCHUNK // SW
RPT = NA // 16
ZR = 125

_f32 = jnp.float32


def _full_spec(shape):
    n = len(shape)
    return pl.BlockSpec(shape, lambda i: (0,) * n)


def _sel(shape, pairs):
    m = np.zeros(shape, np.float32)
    for r, c in pairs:
        m[r, c] = 1.0
    return m

_EB = [np.asarray(_sel((1, C * NB), [(0, 4 * c + k) for c in range(C)]))
       for k in range(NB)]
_GS = _sel((C * NB, C), [(4 * c + k, c) for c in range(C) for k in range(NB)])
_S2_48 = _sel((80, 48), [(16 + c, 16 * x + c) for x in range(3) for c in range(C)])
_S4_48 = _sel((80, 48), [(48 + c, 16 * x + c) for x in range(3) for c in range(C)])
_S3_80 = _sel((80, 80), [(32 + c, 16 * x + c) for x in range(5) for c in range(C)])
_S5_80 = _sel((80, 80), [(64 + c, 16 * x + c) for x in range(5) for c in range(C)])
_E3F = _sel((3, 128), [(x, 16 * x + c) for x in range(3) for c in range(C)])
_M2EF = np.zeros((3, 128), np.float32)
for _c in range(C):
    _M2EF[:, 48 + _c] = [2.0 / 3.0, -1.0 / 3.0, -1.0 / 3.0]
    _M2EF[:, 64 + _c] = [-1.0 / 3.0, 2.0 / 3.0, -1.0 / 3.0]
_A3 = _sel((3, 3), [(0, 0), (0, 1), (1, 2)])
_B3 = _sel((3, 3), [(1, 0), (2, 1), (2, 2)])
_M3EF = _sel((3, 128),
             [(j, 48 + 16 * (2 + j) + c) for j in range(3) for c in range(C)])
_S23F = _sel((80, 128),
             [(16 + c, 16 * x + c) for x in range(3) for c in range(C)]
             + [(32 + c, 48 + 16 * x + c) for x in range(5) for c in range(C)])
_S45F = _sel((80, 128),
             [(48 + c, 16 * x + c) for x in range(3) for c in range(C)]
             + [(64 + c, 48 + 16 * x + c) for x in range(5) for c in range(C)])
_R5 = _sel((80, 16), [(16 * x + c, c) for x in range(5) for c in range(C)])
_R3 = _sel((48, 16), [(16 * x + c, c) for x in range(3) for c in range(C)])
_S2P = _sel((48, 48), [(16 + c, 16 * x + c) for x in range(3) for c in range(C)])
_S3P = _sel((48, 80), [(32 + c, 16 * x + c) for x in range(5) for c in range(C)])
_SEL3 = _sel((128, 48), [(c, c) for c in range(48)])
_SEL5 = _sel((128, 80), [(48 + c, c) for c in range(80)])


def _ssq(diff):
    bs = 32000

    def body(d_ref, o_ref):
        @pl.when(pl.program_id(0) == 0)
        def _():
            o_ref[0, 0] = 0.0

        d = d_ref[...]
        o_ref[0, 0] += jnp.sum(d * d)

    return pl.pallas_call(
        body,
        grid=(NP // bs,),
        in_specs=[pl.BlockSpec((bs, 3), lambda i: (i, 0))],
        out_specs=pl.BlockSpec((1, 1), lambda i: (0, 0),
                               memory_space=pltpu.SMEM),
        out_shape=jax.ShapeDtypeStruct((1, 1), _f32),
    )(diff)


def _onehot(elems2d, w1a, w1b):
    def body(e_ref, wa_ref, wb_ref, t_ref, u_ref, v_ref):
        e = e_ref[...]
        cols = [(e == t).astype(_f32) for t in (1, 6, 7, 8)]
        cols.append(jnp.zeros((e.shape[0], 12), _f32))
        t = jnp.concatenate(cols, axis=1)
        t_ref[...] = t
        u_ref[...] = jnp.dot(t, wa_ref[...], preferred_element_type=_f32)
        v_ref[...] = jnp.dot(t, wb_ref[...], preferred_element_type=_f32)

    return pl.pallas_call(
        body,
        grid=(NA // BA,),
        in_specs=[pl.BlockSpec((BA, 1), lambda i: (i, 0)),
                  _full_spec(w1a.shape), _full_spec(w1b.shape)],
        out_specs=[pl.BlockSpec((BA, 16), lambda i: (i, 0))] * 3,
        out_shape=[jax.ShapeDtypeStruct((NA, 16), _f32)] * 3,
    )(elems2d, w1a, w1b)


def _make_gatherx16():
    mesh = plsc.VectorSubcoreMesh(core_axis_name="c", subcore_axis_name="s")

    @functools.partial(
        pl.kernel,
        mesh=mesh,
        out_type=jax.ShapeDtypeStruct((NP, 16), _f32),
        scratch_types=(
            pltpu.VMEM((GW,), jnp.int32),
            pltpu.VMEM((GW,), jnp.int32),
            pltpu.VMEM((GW, 16), _f32),
            pltpu.VMEM((GW, 16), _f32),
            pltpu.SemaphoreType.DMA,
            pltpu.SemaphoreType.DMA,
        ),
        compiler_params=pltpu.CompilerParams(use_tc_tiling_on_sc=False),
        name="pair_gatherx16",
    )
    def k(u, v, ii, ij, xw, iiv, ijv, r1i, r1j, s1, s2):
        wid = lax.axis_index("s") * 2 + lax.axis_index("c")
        base = wid * PCHUNK

        def body(w, carry):
            start = base + w * GW
            pltpu.sync_copy(ii.at[pl.ds(start, GW)], iiv)
            pltpu.sync_copy(ij.at[pl.ds(start, GW)], ijv)
            c1 = pltpu.async_copy(u.at[iiv], r1i, s1)
            c2 = pltpu.async_copy(v.at[ijv], r1j, s2)
            c1.wait()
            c2.wait()

            def addrow(r4, carry2):
                for dr in range(4):
                    r = r4 * 4 + dr
                    r1i[r, :] = r1i[r, :] + r1j[r, :]
                return carry2

            lax.fori_loop(0, GW // 4, addrow, 0)
            o1 = pltpu.async_copy(r1i, xw.at[pl.ds(start, GW)], s1)
            o1.wait()
            return carry

        lax.fori_loop(0, NWIN_G, body, 0)

    return k


def _make_gather3():
    mesh = plsc.VectorSubcoreMesh(core_axis_name="c", subcore_axis_name="s")

    @functools.partial(
        pl.kernel,
        mesh=mesh,
        out_type=(
            jax.ShapeDtypeStruct((NP, 16), _f32),
            jax.ShapeDtypeStruct((NP, 128), _f32),
        ),
        scratch_types=(
            pltpu.VMEM((GW,), jnp.int32),
            pltpu.VMEM((GW,), jnp.int32),
            pltpu.VMEM((GW, 16), _f32),
            pltpu.VMEM((GW, 16), _f32),
            pltpu.VMEM((GW, 128), _f32),
            pltpu.VMEM((GW, 128), _f32),
        ) + (pltpu.SemaphoreType.DMA,) * 4,
        compiler_params=pltpu.CompilerParams(use_tc_tiling_on_sc=False),
        name="pair_gather3",
    )
    def k(u, v, t35i, t35j, ii, ij, xw, g35,
          iiv, ijv, r1i, r1j, r35i, r35j, s1, s2, s3, s4):
        wid = lax.axis_index("s") * 2 + lax.axis_index("c")
        base = wid * PCHUNK

        def body(w, carry):
            start = base + w * GW
            pltpu.sync_copy(ii.at[pl.ds(start, GW)], iiv)
            pltpu.sync_copy(ij.at[pl.ds(start, GW)], ijv)
            cs = [
                pltpu.async_copy(u.at[iiv], r1i, s1),
                pltpu.async_copy(v.at[ijv], r1j, s2),
                pltpu.async_copy(t35i.at[iiv], r35i, s3),
                pltpu.async_copy(t35j.at[ijv], r35j, s4),
            ]
            for cdesc in cs:
                cdesc.wait()

            def addrow(r4, carry2):
                for dr in range(4):
                    r = r4 * 4 + dr
                    r1i[r, :] = r1i[r, :] + r1j[r, :]
                    for cc in range(8):
                        sl = pl.ds(cc * 16, 16)
                        r35i[r, sl] = r35i[r, sl] + r35j[r, sl]
                return carry2

            lax.fori_loop(0, GW // 4, addrow, 0)
            sl = pl.ds(start, GW)
            o1 = pltpu.async_copy(r1i, xw.at[sl], s1)
            o2 = pltpu.async_copy(r35i, g35.at[sl], s3)
            o1.wait()
            o2.wait()
            return carry

        lax.fori_loop(0, NWIN_G, body, 0)

    return k


def _make_scatter():
    mesh = plsc.VectorSubcoreMesh(core_axis_name="c", subcore_axis_name="s")

    @functools.partial(
        pl.kernel,
        mesh=mesh,
        out_type=(
            jax.ShapeDtypeStruct((2, NA, 16), _f32),
            jax.ShapeDtypeStruct((2, NA, 128), _f32),
        ),
        scratch_types=(
            pltpu.VMEM((SW,), jnp.int32),
            pltpu.VMEM((SW, 16), _f32),
            pltpu.VMEM((ZR, 16), _f32),
            pltpu.VMEM_SHARED((NA, 16), _f32),
        ),
        compiler_params=pltpu.CompilerParams(use_tc_tiling_on_sc=False),
        name="pair_scatter",
    )
    def k(v1, v35, ii, o1, o35, idxv, vv, zv, tbl):
        c = lax.axis_index("c")
        s = lax.axis_index("s")
        wid = s * 2 + c
        base = wid * PCHUNK
        row0 = s * RPT

        def zfill(i, carry):
            zv[i, :] = jnp.zeros((16,), _f32)
            return carry

        lax.fori_loop(0, ZR, zfill, 0)

        chunks = ([(v1, o1, 0)]
                  + [(v35, o35, 16 * x) for x in range(8)])
        for vref, oref, co in chunks:

            def zero_body(i, carry):
                pltpu.sync_copy(zv, tbl.at[pl.ds(row0 + i * ZR, ZR)])
                return carry

            lax.fori_loop(0, RPT // ZR, zero_body, 0)
            plsc.subcore_barrier()

            def win(w, carry):
                start = base + w * SW
                pltpu.sync_copy(ii.at[pl.ds(start, SW)], idxv)
                pltpu.sync_copy(vref.at[pl.ds(start, SW), pl.ds(co, 16)], vv)
                pltpu.sync_copy(vv, tbl.at[idxv], add=True)
                return carry

            lax.fori_loop(0, NWIN_S, win, 0)
            plsc.subcore_barrier()
            pltpu.sync_copy(
                tbl.at[pl.ds(row0, RPT)],
                oref.at[c, pl.ds(row0, RPT), pl.ds(co, 16)],
            )

    return k


def _pair_pallas(first, xw, g35, dist2, diff, ssq, wts):
    names_common = ["b1", "w2", "b2", "iw1", "iw2",
                    "eb0", "eb1", "eb2", "eb3", "gs",
                    "e3f", "m2ef", "a3", "b3", "m3ef", "s23f"]
    names = names_common + ([] if first else ["s45f"])
    warrs = [wts[n] for n in names]

    def body(*refs):
        if first:
            xw_ref, d_ref, df_ref, sq_ref = refs[:4]
            wrefs = refs[4:4 + len(names)]
            v1_ref, v35_ref = refs[4 + len(names):]
            g35_ref = None
        else:
            xw_ref, g35_ref, d_ref, df_ref, sq_ref = refs[:5]
            wrefs = refs[5:5 + len(names)]
            v1_ref, v35_ref = refs[5 + len(names):]
        w = dict(zip(names, wrefs))
        dot = functools.partial(jnp.dot, preferred_element_type=_f32)

        inv = lax.rsqrt(sq_ref[0, 0])
        nd = df_ref[...] * inv
        nd2 = nd * nd
        cross = dot(nd, w["a3"][...]) * dot(nd, w["b3"][...])
        fgeom = (dot(nd, w["e3f"][...]) + dot(nd2, w["m2ef"][...])
                 + dot(cross, w["m3ef"][...]))

        fc = 0.5 * (jnp.cos(_f32(np.pi / RC) * d_ref[...]) + 1.0)
        t2 = fc * fc
        t3 = t2 * fc
        t4 = t2 * t2
        basis_exp = (dot(fc, w["eb0"][...]) + dot(t2, w["eb1"][...])
                     + dot(t3, w["eb2"][...]) + dot(t4, w["eb3"][...]))

        h = jnp.tanh(xw_ref[...] + w["b1"][...])
        h = jnp.tanh(dot(h, w["w2"][...]) + w["b2"][...])
        i1 = dot(h * basis_exp, w["gs"][...])
        i1 = jnp.tanh(dot(i1, w["iw1"][...]))
        i1 = jnp.tanh(dot(i1, w["iw2"][...]))

        v1_ref[...] = i1[:, 0:16]
        v35 = fgeom * dot(i1, w["s23f"][...])
        if not first:
            v35 = v35 + g35_ref[...] * dot(i1, w["s45f"][...])
        v35_ref[...] = v35

    if first:
        arrays = [xw, dist2, diff, ssq] + warrs
        in_specs = [
            pl.BlockSpec((BP, 16), lambda i: (i, 0)),
            pl.BlockSpec((BP, 1), lambda i: (i, 0)),
            pl.BlockSpec((BP, 3), lambda i: (i, 0)),
            pl.BlockSpec((1, 1), lambda i: (0, 0), memory_space=pltpu.SMEM),
        ] + [_full_spec(a.shape) for a in warrs]
    else:
        arrays = [xw, g35, dist2, diff, ssq] + warrs
        in_specs = [
            pl.BlockSpec((BP, 16), lambda i: (i, 0)),
            pl.BlockSpec((BP, 128), lambda i: (i, 0)),
            pl.BlockSpec((BP, 1), lambda i: (i, 0)),
            pl.BlockSpec((BP, 3), lambda i: (i, 0)),
            pl.BlockSpec((1, 1), lambda i: (0, 0), memory_space=pltpu.SMEM),
        ] + [_full_spec(a.shape) for a in warrs]

    return pl.pallas_call(
        body,
        grid=(NP // BP,),
        in_specs=in_specs,
        out_specs=[
            pl.BlockSpec((BP, 16), lambda i: (i, 0)),
            pl.BlockSpec((BP, 128), lambda i: (i, 0)),
        ],
        out_shape=[
            jax.ShapeDtypeStruct((NP, 16), _f32),
            jax.ShapeDtypeStruct((NP, 128), _f32),
        ],
    )(*arrays)


def _atom_pallas(first, last, o1, o35, prevs, out_prev, wts):
    names = ["sel3", "sel5", "d1i", "d1j", "r5", "d2i", "d2j", "r3",
             "pw1a", "pw1b", "pw1c", "pb1", "pw2", "pb2",
             "s2p", "s3p", "p3b", "p5b",
             "ow1", "ob1", "ow2", "ob2", "outw", "outb"]
    if first:
        names.append("res1p")
    if not last:
        names += ["x3i", "x3j", "x5i", "x5j", "nw1a", "nw1b"]
    warrs = [wts[n] for n in names]
    nin_prev = 1 if first else 3

    def body(*refs):
        pa1, pb1_, pa35, pb35 = refs[:4]
        prefs = refs[4:4 + nin_prev]
        op_ref = refs[4 + nin_prev]
        wrefs = refs[5 + nin_prev:5 + nin_prev + len(names)]
        out_refs = refs[5 + nin_prev + len(names):]
        w = dict(zip(names, wrefs))
        dot = functools.partial(jnp.dot, preferred_element_type=_f32)

        p1n = pa1[0] + pb1_[0]
        pn35 = pa35[0] + pb35[0]
        p3n = dot(pn35, w["sel3"][...])
        p5n = dot(pn35, w["sel5"][...])

        a5 = dot(p5n, w["d1i"][...])
        b5 = dot(p5n, w["d1j"][...])
        dot1 = dot(a5 * b5, w["r5"][...])
        a3 = dot(p3n, w["d2i"][...])
        b3 = dot(p3n, w["d2j"][...])
        dot2 = dot(a3 * b3, w["r3"][...])

        p1t = jnp.tanh(dot(dot1, w["pw1a"][...]) + dot(dot2, w["pw1b"][...])
                       + dot(p1n, w["pw1c"][...]) + w["pb1"][...])
        p1t = jnp.tanh(dot(p1t, w["pw2"][...]) + w["pb2"][...])
        p1t1 = p1t[:, 0:16]
        p3t = dot(p3n * dot(p1t, w["s2p"][...]), w["p3b"][...])
        p5t = dot(p5n * dot(p1t, w["s3p"][...]), w["p5b"][...])

        h = dot(p1t1, w["ow1"][...]) + w["ob1"][...]
        h = dot(h, w["ow2"][...]) + w["ob2"][...]
        o_ref = out_refs[0]
        o_ref[...] = op_ref[...] + dot(h, w["outw"][...]) + w["outb"][...]

        if first:
            p1 = dot(prefs[0][...], w["res1p"][...]) + p1t1
            p3 = p3t
            p5 = p5t
        else:
            p1 = prefs[0][...] + p1t1
            p3 = prefs[1][...] + p3t
            p5 = prefs[2][...] + p5t
        if not last:
            (p1_ref, p3_ref, p5_ref, u_ref, v_ref,
             t35i_ref, t35j_ref) = out_refs[1:]
            p1_ref[...] = p1
            p3_ref[...] = p3
            p5_ref[...] = p5
            u_ref[...] = dot(p1, w["nw1a"][...])
            v_ref[...] = dot(p1, w["nw1b"][...])
            t35i_ref[...] = jnp.concatenate(
                [dot(p3, w["x3i"][...]), dot(p5, w["x5i"][...])], axis=1)
            t35j_ref[...] = jnp.concatenate(
                [dot(p3, w["x3j"][...]), dot(p5, w["x5j"][...])], axis=1)

    arrays = [o1, o1, o35, o35] + list(prevs) + [out_prev] + warrs
    in_specs = [
        pl.BlockSpec((1, BA, 16), lambda i: (0, i, 0)),
        pl.BlockSpec((1, BA, 16), lambda i: (1, i, 0)),
        pl.BlockSpec((1, BA, 128), lambda i: (0, i, 0)),
        pl.BlockSpec((1, BA, 128), lambda i: (1, i, 0)),
    ]
    for p in prevs:
        in_specs.append(pl.BlockSpec((BA, p.shape[1]), lambda i: (i, 0)))
    in_specs.append(pl.BlockSpec((BA, 1), lambda i: (i, 0)))
    in_specs += [_full_spec(a.shape) for a in warrs]

    out_shapes = [jax.ShapeDtypeStruct((NA, 1), _f32)]
    out_specs = [pl.BlockSpec((BA, 1), lambda i: (i, 0))]
    if not last:
        for cols in (16, 48, 80, 16, 16, 128, 128):
            out_shapes.append(jax.ShapeDtypeStruct((NA, cols), _f32))
            out_specs.append(pl.BlockSpec((BA, cols), lambda i: (i, 0)))

    return pl.pallas_call(
        body,
        grid=(NA // BA,),
        in_specs=in_specs,
        out_specs=out_specs,
        out_shape=out_shapes,
    )(*arrays)


def _w1split(params, b):
    bp = params["block%d" % b]
    w1 = bp["pi1"][0][0]
    cin = NE if b == 0 else C
    w1a = w1[:cin]
    w1b = w1[cin:]
    if b == 0:
        w1a = jnp.zeros((16, C), _f32).at[:cin].set(w1a)
        w1b = jnp.zeros((16, C), _f32).at[:cin].set(w1b)
    return w1a, w1b


def kernel(ind_1, elems, coord, ind_2, dist, diff, params):
    ii = ind_2[:, 0]
    ij = ind_2[:, 1]
    dist2 = dist[:, None]
    ssq = _ssq(diff)
    w1a0, w1b0 = _w1split(params, 0)
    t0, u, v = _onehot(elems[:, None].astype(jnp.int32), w1a0, w1b0)

    consts = {
        "eb0": _EB[0], "eb1": _EB[1], "eb2": _EB[2], "eb3": _EB[3],
        "gs": _GS, "e3f": _E3F, "m2ef": _M2EF, "a3": _A3, "b3": _B3,
        "m3ef": _M3EF, "s23f": _S23F, "s45f": _S45F,
    }
    consts = {k: jnp.asarray(v) for k, v in consts.items()}
    aconsts = {"r5": jnp.asarray(_R5), "r3": jnp.asarray(_R3),
               "s2p": jnp.asarray(_S2P), "s3p": jnp.asarray(_S3P),
               "sel3": jnp.asarray(_SEL3), "sel5": jnp.asarray(_SEL5)}
    eye3 = jnp.eye(3, dtype=_f32)
    eye5 = jnp.eye(5, dtype=_f32)

    gatherx16 = _make_gatherx16()
    gather3 = _make_gather3()
    scatter = _make_scatter()

    out_acc = jnp.zeros((NA, 1), _f32)
    prevs = (t0,)
    t35i = t35j = None
    for b in range(DEPTH):
        bp = params["block%d" % b]
        first = b == 0
        last = b == DEPTH - 1
        if first:
            xw = gatherx16(u, v, ii, ij)
            g35 = None
        else:
            xw, g35 = gather3(u, v, t35i, t35j, ii, ij)
        (w1, b1), (w2, b2) = bp["pi1"]
        iw1, iw2 = bp["ii1"]
        wts = dict(consts)
        wts.update(b1=b1[None, :], w2=w2, b2=b2[None, :],
                   iw1=iw1, iw2=iw2)
        v1, v35 = _pair_pallas(first, xw, g35, dist2, diff, ssq, wts)
        o1, o35 = scatter(v1, v35, ii)

        (pw1, pb1), (pw2, pb2) = bp["pp1"]
        (ow1, ob1), (ow2, ob2) = bp["out_ff"]
        awts = dict(aconsts)
        awts.update(
            d1i=jnp.kron(eye5, bp["dot1_wi"]), d1j=jnp.kron(eye5, bp["dot1_wj"]),
            d2i=jnp.kron(eye3, bp["dot2_wi"]), d2j=jnp.kron(eye3, bp["dot2_wj"]),
            pw1a=pw1[0:16], pw1b=pw1[16:32], pw1c=pw1[32:48],
            pb1=pb1[None, :], pw2=pw2, pb2=pb2[None, :],
            p3b=jnp.kron(eye3, bp["pp3_W"]), p5b=jnp.kron(eye5, bp["pp5_W"]),
            ow1=ow1, ob1=ob1[None, :], ow2=ow2, ob2=ob2[None, :],
            outw=bp["out_W"], outb=bp["out_b"][None, :])
        if first:
            awts["res1p"] = jnp.zeros((16, C), _f32).at[:NE].set(
                params["res1_W"])
        if not last:
            nbp = params["block%d" % (b + 1)]
            nw1a, nw1b = _w1split(params, b + 1)
            awts.update(
                x3i=jnp.kron(eye3, nbp["pix3_wi"]),
                x3j=jnp.kron(eye3, nbp["pix3_wj"]),
                x5i=jnp.kron(eye5, nbp["pix5_wi"]),
                x5j=jnp.kron(eye5, nbp["pix5_wj"]),
                nw1a=nw1a, nw1b=nw1b)
        outs = _atom_pallas(first, last, o1, o35, prevs, out_acc, awts)
        if last:
            out_acc = outs[0]
        else:
            out_acc = outs[0]
            prevs = (outs[1], outs[2], outs[3])
            u, v, t35i, t35j = outs[4], outs[5], outs[6], outs[7]
    return out_acc[:, 0]


# speedup vs baseline: 40.1994x; 1.0846x over previous
import functools

import numpy as np
import jax
import jax.numpy as jnp
from jax import lax
from jax.experimental import pallas as pl
from jax.experimental.pallas import tpu as pltpu
import jax.experimental.pallas.tpu_sc as plsc

C = 16
NB = 4
NE = 4
DEPTH = 4
RC = 4.0
NA = 50000
NP = 800000

BP = 4000
BA = 2000
NWORK = 32
PCHUNK = NP // NWORK
GW = 200
NWIN_G = PCHUNK // GW
SW = 1000
NWIN_S = PCHUNK // SW
RPT = NA // 16
ZR = 125

_f32 = jnp.float32


def _full_spec(shape):
    n = len(shape)
    return pl.BlockSpec(shape, lambda i: (0,) * n)


def _sel(shape, pairs):
    m = np.zeros(shape, np.float32)
    for r, c in pairs:
        m[r, c] = 1.0
    return m

_EB = [np.asarray(_sel((1, C * NB), [(0, 4 * c + k) for c in range(C)]))
       for k in range(NB)]
_GS = _sel((C * NB, C), [(4 * c + k, c) for c in range(C) for k in range(NB)])
_S2_48 = _sel((80, 48), [(16 + c, 16 * x + c) for x in range(3) for c in range(C)])
_S4_48 = _sel((80, 48), [(48 + c, 16 * x + c) for x in range(3) for c in range(C)])
_S3_80 = _sel((80, 80), [(32 + c, 16 * x + c) for x in range(5) for c in range(C)])
_S5_80 = _sel((80, 80), [(64 + c, 16 * x + c) for x in range(5) for c in range(C)])
_E3F = _sel((3, 128), [(x, 16 * x + c) for x in range(3) for c in range(C)])
_M2EF = np.zeros((3, 128), np.float32)
for _c in range(C):
    _M2EF[:, 48 + _c] = [2.0 / 3.0, -1.0 / 3.0, -1.0 / 3.0]
    _M2EF[:, 64 + _c] = [-1.0 / 3.0, 2.0 / 3.0, -1.0 / 3.0]
_A3 = _sel((3, 3), [(0, 0), (0, 1), (1, 2)])
_B3 = _sel((3, 3), [(1, 0), (2, 1), (2, 2)])
_M3EF = _sel((3, 128),
             [(j, 48 + 16 * (2 + j) + c) for j in range(3) for c in range(C)])
_S23F = _sel((80, 128),
             [(16 + c, 16 * x + c) for x in range(3) for c in range(C)]
             + [(32 + c, 48 + 16 * x + c) for x in range(5) for c in range(C)])
_S45F = _sel((80, 128),
             [(48 + c, 16 * x + c) for x in range(3) for c in range(C)]
             + [(64 + c, 48 + 16 * x + c) for x in range(5) for c in range(C)])
_R5 = _sel((80, 16), [(16 * x + c, c) for x in range(5) for c in range(C)])
_R3 = _sel((48, 16), [(16 * x + c, c) for x in range(3) for c in range(C)])
_S2P = _sel((48, 48), [(16 + c, 16 * x + c) for x in range(3) for c in range(C)])
_S3P = _sel((48, 80), [(32 + c, 16 * x + c) for x in range(5) for c in range(C)])
_SEL3 = _sel((128, 48), [(c, c) for c in range(48)])
_SEL5 = _sel((128, 80), [(48 + c, c) for c in range(80)])


def _ssq(diff):
    bs = 32000

    def body(d_ref, o_ref):
        @pl.when(pl.program_id(0) == 0)
        def _():
            o_ref[0, 0] = 0.0

        d = d_ref[...]
        o_ref[0, 0] += jnp.sum(d * d)

    return pl.pallas_call(
        body,
        grid=(NP // bs,),
        in_specs=[pl.BlockSpec((bs, 3), lambda i: (i, 0))],
        out_specs=pl.BlockSpec((1, 1), lambda i: (0, 0),
                               memory_space=pltpu.SMEM),
        out_shape=jax.ShapeDtypeStruct((1, 1), _f32),
    )(diff)


def _onehot(elems2d, w1a, w1b):
    def body(e_ref, wa_ref, wb_ref, t_ref, u_ref, v_ref):
        e = e_ref[...]
        cols = [(e == t).astype(_f32) for t in (1, 6, 7, 8)]
        cols.append(jnp.zeros((e.shape[0], 12), _f32))
        t = jnp.concatenate(cols, axis=1)
        t_ref[...] = t
        u_ref[...] = jnp.dot(t, wa_ref[...], preferred_element_type=_f32)
        v_ref[...] = jnp.dot(t, wb_ref[...], preferred_element_type=_f32)

    return pl.pallas_call(
        body,
        grid=(NA // BA,),
        in_specs=[pl.BlockSpec((BA, 1), lambda i: (i, 0)),
                  _full_spec(w1a.shape), _full_spec(w1b.shape)],
        out_specs=[pl.BlockSpec((BA, 16), lambda i: (i, 0))] * 3,
        out_shape=[jax.ShapeDtypeStruct((NA, 16), _f32)] * 3,
    )(elems2d, w1a, w1b)


def _make_gatherx16():
    mesh = plsc.VectorSubcoreMesh(core_axis_name="c", subcore_axis_name="s")

    @functools.partial(
        pl.kernel,
        mesh=mesh,
        out_type=(
            jax.ShapeDtypeStruct((NP, 16), _f32),
            jax.ShapeDtypeStruct((NP, 16), _f32),
        ),
        scratch_types=(
            pltpu.VMEM((GW,), jnp.int32),
            pltpu.VMEM((GW,), jnp.int32),
            pltpu.VMEM((GW, 16), _f32),
            pltpu.VMEM((GW, 16), _f32),
            pltpu.SemaphoreType.DMA,
            pltpu.SemaphoreType.DMA,
        ),
        compiler_params=pltpu.CompilerParams(use_tc_tiling_on_sc=False),
        name="pair_gatherx16",
    )
    def k(u, v, ii, ij, xwi, xwj, iiv, ijv, r1i, r1j, s1, s2):
        wid = lax.axis_index("s") * 2 + lax.axis_index("c")
        base = wid * PCHUNK

        def body(w, carry):
            start = base + w * GW
            pltpu.sync_copy(ii.at[pl.ds(start, GW)], iiv)
            pltpu.sync_copy(ij.at[pl.ds(start, GW)], ijv)
            c1 = pltpu.async_copy(u.at[iiv], r1i, s1)
            c2 = pltpu.async_copy(v.at[ijv], r1j, s2)
            c1.wait()
            c2.wait()
            o1 = pltpu.async_copy(r1i, xwi.at[pl.ds(start, GW)], s1)
            o2 = pltpu.async_copy(r1j, xwj.at[pl.ds(start, GW)], s2)
            o1.wait()
            o2.wait()
            return carry

        lax.fori_loop(0, NWIN_G, body, 0)

    return k


def _make_gather3():
    mesh = plsc.VectorSubcoreMesh(core_axis_name="c", subcore_axis_name="s")

    @functools.partial(
        pl.kernel,
        mesh=mesh,
        out_type=(
            jax.ShapeDtypeStruct((NP, 16), _f32),
            jax.ShapeDtypeStruct((NP, 16), _f32),
            jax.ShapeDtypeStruct((NP, 128), _f32),
            jax.ShapeDtypeStruct((NP, 128), _f32),
        ),
        scratch_types=(
            pltpu.VMEM((GW,), jnp.int32),
            pltpu.VMEM((GW,), jnp.int32),
            pltpu.VMEM((GW, 16), _f32),
            pltpu.VMEM((GW, 16), _f32),
            pltpu.VMEM((GW, 128), _f32),
            pltpu.VMEM((GW, 128), _f32),
        ) + (pltpu.SemaphoreType.DMA,) * 4,
        compiler_params=pltpu.CompilerParams(use_tc_tiling_on_sc=False),
        name="pair_gather3",
    )
    def k(u, v, t35i, t35j, ii, ij, xwi, xwj, g35i, g35j,
          iiv, ijv, r1i, r1j, r35i, r35j, s1, s2, s3, s4):
        wid = lax.axis_index("s") * 2 + lax.axis_index("c")
        base = wid * PCHUNK

        def body(w, carry):
            start = base + w * GW
            pltpu.sync_copy(ii.at[pl.ds(start, GW)], iiv)
            pltpu.sync_copy(ij.at[pl.ds(start, GW)], ijv)
            cs = [
                pltpu.async_copy(u.at[iiv], r1i, s1),
                pltpu.async_copy(v.at[ijv], r1j, s2),
                pltpu.async_copy(t35i.at[iiv], r35i, s3),
                pltpu.async_copy(t35j.at[ijv], r35j, s4),
            ]
            for cdesc in cs:
                cdesc.wait()
            sl = pl.ds(start, GW)
            os = [
                pltpu.async_copy(r1i, xwi.at[sl], s1),
                pltpu.async_copy(r1j, xwj.at[sl], s2),
                pltpu.async_copy(r35i, g35i.at[sl], s3),
                pltpu.async_copy(r35j, g35j.at[sl], s4),
            ]
            for odesc in os:
                odesc.wait()
            return carry

        lax.fori_loop(0, NWIN_G, body, 0)

    return k


def _make_scatter():
    mesh = plsc.VectorSubcoreMesh(core_axis_name="c", subcore_axis_name="s")

    @functools.partial(
        pl.kernel,
        mesh=mesh,
        out_type=(
            jax.ShapeDtypeStruct((2, NA, 16), _f32),
            jax.ShapeDtypeStruct((2, NA, 128), _f32),
        ),
        scratch_types=(
            pltpu.VMEM((SW,), jnp.int32),
            pltpu.VMEM((SW, 16), _f32),
            pltpu.VMEM((ZR, 16), _f32),
            pltpu.VMEM_SHARED((NA, 16), _f32),
        ),
        compiler_params=pltpu.CompilerParams(use_tc_tiling_on_sc=False),
        name="pair_scatter",
    )
    def k(v1, v35, ii, o1, o35, idxv, vv, zv, tbl):
        c = lax.axis_index("c")
        s = lax.axis_index("s")
        wid = s * 2 + c
        base = wid * PCHUNK
        row0 = s * RPT

        def zfill(i, carry):
            zv[i, :] = jnp.zeros((16,), _f32)
            return carry

        lax.fori_loop(0, ZR, zfill, 0)

        chunks = ([(v1, o1, 0)]
                  + [(v35, o35, 16 * x) for x in range(8)])
        for vref, oref, co in chunks:

            def zero_body(i, carry):
                pltpu.sync_copy(zv, tbl.at[pl.ds(row0 + i * ZR, ZR)])
                return carry

            lax.fori_loop(0, RPT // ZR, zero_body, 0)
            plsc.subcore_barrier()

            def win(w, carry):
                start = base + w * SW
                pltpu.sync_copy(ii.at[pl.ds(start, SW)], idxv)
                pltpu.sync_copy(vref.at[pl.ds(start, SW), pl.ds(co, 16)], vv)
                pltpu.sync_copy(vv, tbl.at[idxv], add=True)
                return carry

            lax.fori_loop(0, NWIN_S, win, 0)
            plsc.subcore_barrier()
            pltpu.sync_copy(
                tbl.at[pl.ds(row0, RPT)],
                oref.at[c, pl.ds(row0, RPT), pl.ds(co, 16)],
            )

    return k


def _pair_pallas(first, xws, g35s, dist2, diff, ssq, wts):
    names_common = ["b1", "w2", "b2", "iw1", "iw2",
                    "eb0", "eb1", "eb2", "eb3", "gs",
                    "e3f", "m2ef", "a3", "b3", "m3ef", "s23f"]
    names = names_common + ([] if first else ["s45f"])
    warrs = [wts[n] for n in names]

    def body(*refs):
        if first:
            xwi_ref, xwj_ref, d_ref, df_ref, sq_ref = refs[:5]
            wrefs = refs[5:5 + len(names)]
            v1_ref, v35_ref = refs[5 + len(names):]
            g35i_ref = g35j_ref = None
        else:
            (xwi_ref, xwj_ref, g35i_ref, g35j_ref,
             d_ref, df_ref, sq_ref) = refs[:7]
            wrefs = refs[7:7 + len(names)]
            v1_ref, v35_ref = refs[7 + len(names):]
        w = dict(zip(names, wrefs))
        dot = functools.partial(jnp.dot, preferred_element_type=_f32)

        inv = lax.rsqrt(sq_ref[0, 0])
        nd = df_ref[...] * inv
        nd2 = nd * nd
        cross = dot(nd, w["a3"][...]) * dot(nd, w["b3"][...])
        fgeom = (dot(nd, w["e3f"][...]) + dot(nd2, w["m2ef"][...])
                 + dot(cross, w["m3ef"][...]))

        fc = 0.5 * (jnp.cos(_f32(np.pi / RC) * d_ref[...]) + 1.0)
        t2 = fc * fc
        t3 = t2 * fc
        t4 = t2 * t2
        basis_exp = (dot(fc, w["eb0"][...]) + dot(t2, w["eb1"][...])
                     + dot(t3, w["eb2"][...]) + dot(t4, w["eb3"][...]))

        h = jnp.tanh(xwi_ref[...] + xwj_ref[...] + w["b1"][...])
        h = jnp.tanh(dot(h, w["w2"][...]) + w["b2"][...])
        i1 = dot(h * basis_exp, w["gs"][...])
        i1 = jnp.tanh(dot(i1, w["iw1"][...]))
        i1 = jnp.tanh(dot(i1, w["iw2"][...]))

        v1_ref[...] = i1[:, 0:16]
        v35 = fgeom * dot(i1, w["s23f"][...])
        if not first:
            v35 = v35 + ((g35i_ref[...] + g35j_ref[...])
                         * dot(i1, w["s45f"][...]))
        v35_ref[...] = v35

    if first:
        arrays = list(xws) + [dist2, diff, ssq] + warrs
        in_specs = [
            pl.BlockSpec((BP, 16), lambda i: (i, 0)),
            pl.BlockSpec((BP, 16), lambda i: (i, 0)),
            pl.BlockSpec((BP, 1), lambda i: (i, 0)),
            pl.BlockSpec((BP, 3), lambda i: (i, 0)),
            pl.BlockSpec((1, 1), lambda i: (0, 0), memory_space=pltpu.SMEM),
        ] + [_full_spec(a.shape) for a in warrs]
    else:
        arrays = list(xws) + list(g35s) + [dist2, diff, ssq] + warrs
        in_specs = [
            pl.BlockSpec((BP, 16), lambda i: (i, 0)),
            pl.BlockSpec((BP, 16), lambda i: (i, 0)),
            pl.BlockSpec((BP, 128), lambda i: (i, 0)),
            pl.BlockSpec((BP, 128), lambda i: (i, 0)),
            pl.BlockSpec((BP, 1), lambda i: (i, 0)),
            pl.BlockSpec((BP, 3), lambda i: (i, 0)),
            pl.BlockSpec((1, 1), lambda i: (0, 0), memory_space=pltpu.SMEM),
        ] + [_full_spec(a.shape) for a in warrs]

    return pl.pallas_call(
        body,
        grid=(NP // BP,),
        in_specs=in_specs,
        out_specs=[
            pl.BlockSpec((BP, 16), lambda i: (i, 0)),
            pl.BlockSpec((BP, 128), lambda i: (i, 0)),
        ],
        out_shape=[
            jax.ShapeDtypeStruct((NP, 16), _f32),
            jax.ShapeDtypeStruct((NP, 128), _f32),
        ],
    )(*arrays)


def _atom_pallas(first, last, o1, o35, prevs, out_prev, wts):
    names = ["sel3", "sel5", "d1i", "d1j", "r5", "d2i", "d2j", "r3",
             "pw1a", "pw1b", "pw1c", "pb1", "pw2", "pb2",
             "s2p", "s3p", "p3b", "p5b",
             "ow1", "ob1", "ow2", "ob2", "outw", "outb"]
    if first:
        names.append("res1p")
    if not last:
        names += ["x3i", "x3j", "x5i", "x5j", "nw1a", "nw1b"]
    warrs = [wts[n] for n in names]
    nin_prev = 1 if first else 3

    def body(*refs):
        pa1, pb1_, pa35, pb35 = refs[:4]
        prefs = refs[4:4 + nin_prev]
        op_ref = refs[4 + nin_prev]
        wrefs = refs[5 + nin_prev:5 + nin_prev + len(names)]
        out_refs = refs[5 + nin_prev + len(names):]
        w = dict(zip(names, wrefs))
        dot = functools.partial(jnp.dot, preferred_element_type=_f32)

        p1n = pa1[0] + pb1_[0]
        pn35 = pa35[0] + pb35[0]
        p3n = dot(pn35, w["sel3"][...])
        p5n = dot(pn35, w["sel5"][...])

        a5 = dot(p5n, w["d1i"][...])
        b5 = dot(p5n, w["d1j"][...])
        dot1 = dot(a5 * b5, w["r5"][...])
        a3 = dot(p3n, w["d2i"][...])
        b3 = dot(p3n, w["d2j"][...])
        dot2 = dot(a3 * b3, w["r3"][...])

        p1t = jnp.tanh(dot(dot1, w["pw1a"][...]) + dot(dot2, w["pw1b"][...])
                       + dot(p1n, w["pw1c"][...]) + w["pb1"][...])
        p1t = jnp.tanh(dot(p1t, w["pw2"][...]) + w["pb2"][...])
        p1t1 = p1t[:, 0:16]
        p3t = dot(p3n * dot(p1t, w["s2p"][...]), w["p3b"][...])
        p5t = dot(p5n * dot(p1t, w["s3p"][...]), w["p5b"][...])

        h = dot(p1t1, w["ow1"][...]) + w["ob1"][...]
        h = dot(h, w["ow2"][...]) + w["ob2"][...]
        o_ref = out_refs[0]
        o_ref[...] = op_ref[...] + dot(h, w["outw"][...]) + w["outb"][...]

        if first:
            p1 = dot(prefs[0][...], w["res1p"][...]) + p1t1
            p3 = p3t
            p5 = p5t
        else:
            p1 = prefs[0][...] + p1t1
            p3 = prefs[1][...] + p3t
            p5 = prefs[2][...] + p5t
        if not last:
            (p1_ref, p3_ref, p5_ref, u_ref, v_ref,
             t35i_ref, t35j_ref) = out_refs[1:]
            p1_ref[...] = p1
            p3_ref[...] = p3
            p5_ref[...] = p5
            u_ref[...] = dot(p1, w["nw1a"][...])
            v_ref[...] = dot(p1, w["nw1b"][...])
            t35i_ref[...] = jnp.concatenate(
                [dot(p3, w["x3i"][...]), dot(p5, w["x5i"][...])], axis=1)
            t35j_ref[...] = jnp.concatenate(
                [dot(p3, w["x3j"][...]), dot(p5, w["x5j"][...])], axis=1)

    arrays = [o1, o1, o35, o35] + list(prevs) + [out_prev] + warrs
    in_specs = [
        pl.BlockSpec((1, BA, 16), lambda i: (0, i, 0)),
        pl.BlockSpec((1, BA, 16), lambda i: (1, i, 0)),
        pl.BlockSpec((1, BA, 128), lambda i: (0, i, 0)),
        pl.BlockSpec((1, BA, 128), lambda i: (1, i, 0)),
    ]
    for p in prevs:
        in_specs.append(pl.BlockSpec((BA, p.shape[1]), lambda i: (i, 0)))
    in_specs.append(pl.BlockSpec((BA, 1), lambda i: (i, 0)))
    in_specs += [_full_spec(a.shape) for a in warrs]

    out_shapes = [jax.ShapeDtypeStruct((NA, 1), _f32)]
    out_specs = [pl.BlockSpec((BA, 1), lambda i: (i, 0))]
    if not last:
        for cols in (16, 48, 80, 16, 16, 128, 128):
            out_shapes.append(jax.ShapeDtypeStruct((NA, cols), _f32))
            out_specs.append(pl.BlockSpec((BA, cols), lambda i: (i, 0)))

    return pl.pallas_call(
        body,
        grid=(NA // BA,),
        in_specs=in_specs,
        out_specs=out_specs,
        out_shape=out_shapes,
    )(*arrays)


def _w1split(params, b):
    bp = params["block%d" % b]
    w1 = bp["pi1"][0][0]
    cin = NE if b == 0 else C
    w1a = w1[:cin]
    w1b = w1[cin:]
    if b == 0:
        w1a = jnp.zeros((16, C), _f32).at[:cin].set(w1a)
        w1b = jnp.zeros((16, C), _f32).at[:cin].set(w1b)
    return w1a, w1b


def kernel(ind_1, elems, coord, ind_2, dist, diff, params):
    ii = ind_2[:, 0]
    ij = ind_2[:, 1]
    dist2 = dist[:, None]
    ssq = _ssq(diff)
    w1a0, w1b0 = _w1split(params, 0)
    t0, u, v = _onehot(elems[:, None].astype(jnp.int32), w1a0, w1b0)

    consts = {
        "eb0": _EB[0], "eb1": _EB[1], "eb2": _EB[2], "eb3": _EB[3],
        "gs": _GS, "e3f": _E3F, "m2ef": _M2EF, "a3": _A3, "b3": _B3,
        "m3ef": _M3EF, "s23f": _S23F, "s45f": _S45F,
    }
    consts = {k: jnp.asarray(v) for k, v in consts.items()}
    aconsts = {"r5": jnp.asarray(_R5), "r3": jnp.asarray(_R3),
               "s2p": jnp.asarray(_S2P), "s3p": jnp.asarray(_S3P),
               "sel3": jnp.asarray(_SEL3), "sel5": jnp.asarray(_SEL5)}
    eye3 = jnp.eye(3, dtype=_f32)
    eye5 = jnp.eye(5, dtype=_f32)

    gatherx16 = _make_gatherx16()
    gather3 = _make_gather3()
    scatter = _make_scatter()

    out_acc = jnp.zeros((NA, 1), _f32)
    prevs = (t0,)
    t35i = t35j = None
    for b in range(DEPTH):
        bp = params["block%d" % b]
        first = b == 0
        last = b == DEPTH - 1
        if first:
            xws = gatherx16(u, v, ii, ij)
            g35s = None
        else:
            xwi, xwj, g35i, g35j = gather3(u, v, t35i, t35j, ii, ij)
            xws = (xwi, xwj)
            g35s = (g35i, g35j)
        (w1, b1), (w2, b2) = bp["pi1"]
        iw1, iw2 = bp["ii1"]
        wts = dict(consts)
        wts.update(b1=b1[None, :], w2=w2, b2=b2[None, :],
                   iw1=iw1, iw2=iw2)
        v1, v35 = _pair_pallas(first, xws, g35s, dist2, diff, ssq, wts)
        o1, o35 = scatter(v1, v35, ii)

        (pw1, pb1), (pw2, pb2) = bp["pp1"]
        (ow1, ob1), (ow2, ob2) = bp["out_ff"]
        awts = dict(aconsts)
        awts.update(
            d1i=jnp.kron(eye5, bp["dot1_wi"]), d1j=jnp.kron(eye5, bp["dot1_wj"]),
            d2i=jnp.kron(eye3, bp["dot2_wi"]), d2j=jnp.kron(eye3, bp["dot2_wj"]),
            pw1a=pw1[0:16], pw1b=pw1[16:32], pw1c=pw1[32:48],
            pb1=pb1[None, :], pw2=pw2, pb2=pb2[None, :],
            p3b=jnp.kron(eye3, bp["pp3_W"]), p5b=jnp.kron(eye5, bp["pp5_W"]),
            ow1=ow1, ob1=ob1[None, :], ow2=ow2, ob2=ob2[None, :],
            outw=bp["out_W"], outb=bp["out_b"][None, :])
        if first:
            awts["res1p"] = jnp.zeros((16, C), _f32).at[:NE].set(
                params["res1_W"])
        if not last:
            nbp = params["block%d" % (b + 1)]
            nw1a, nw1b = _w1split(params, b + 1)
            awts.update(
                x3i=jnp.kron(eye3, nbp["pix3_wi"]),
                x3j=jnp.kron(eye3, nbp["pix3_wj"]),
                x5i=jnp.kron(eye5, nbp["pix5_wi"]),
                x5j=jnp.kron(eye5, nbp["pix5_wj"]),
                nw1a=nw1a, nw1b=nw1b)
        outs = _atom_pallas(first, last, o1, o35, prevs, out_acc, awts)
        if last:
            out_acc = outs[0]
        else:
            out_acc = outs[0]
            prevs = (outs[1], outs[2], outs[3])
            u, v, t35i, t35j = outs[4], outs[5], outs[6], outs[7]
    return out_acc[:, 0]
